# Initial kernel scaffold; baseline (speedup 1.0000x reference)
import jax
import jax.numpy as jnp
from jax.experimental import pallas as pl


def kernel(xyz, points, W1, b1, W2, b2, W3, b3):
    raise NotImplementedError("write your pallas kernel here")


# speedup vs baseline: 10.6945x; 10.6945x over previous
import functools

import numpy as np
import jax
import jax.numpy as jnp
from jax import lax
from jax.experimental import pallas as pl
from jax.experimental.pallas import tpu as pltpu
from jax.experimental.pallas import tpu_sc as plsc

_B, _N, _DP = 8, 4096, 64
_NP = 512
_NS = 32
_CB = 128
_T04 = np.float32(0.2 ** 2)


def _fps_body(xyz_ref, init_ref, cx_ref, cy_ref, cz_ref):
    X = xyz_ref[:, 0, :]
    Y = xyz_ref[:, 1, :]
    Z = xyz_ref[:, 2, :]
    iota = lax.broadcasted_iota(jnp.int32, (_B, _N), 1)
    slot = lax.broadcasted_iota(jnp.int32, (_B, _NP), 1)
    zero = jnp.zeros((_B, _N), jnp.float32)
    zc = jnp.zeros((_B, _NP), jnp.float32)

    def pick(sel):
        px = jnp.sum(jnp.where(sel, X, zero), axis=1, keepdims=True)
        py = jnp.sum(jnp.where(sel, Y, zero), axis=1, keepdims=True)
        pz = jnp.sum(jnp.where(sel, Z, zero), axis=1, keepdims=True)
        return px, py, pz

    init_i = init_ref[...].astype(jnp.int32)
    px, py, pz = pick(iota == init_i)
    cxs = jnp.where(slot == 0, px, zc)
    cys = jnp.where(slot == 0, py, zc)
    czs = jnp.where(slot == 0, pz, zc)
    mask = jnp.ones((_B, _N), jnp.float32)

    def body(i, carry):
        px, py, pz, cxs, cys, czs, mask = carry
        dx = X - px
        dy = Y - py
        dz = Z - pz
        d = jnp.sqrt(dx * dx + dy * dy + dz * dz)
        dm = d * mask
        mx = jnp.max(dm, axis=1, keepdims=True)
        idx = jnp.min(jnp.where(dm == mx, iota, jnp.int32(_N)), axis=1,
                      keepdims=True)
        npx, npy, npz = pick(iota == idx)
        nmask = jnp.minimum(dm * mask * jnp.float32(1e11), mask)
        w = slot == (i + 1)
        cxs = jnp.where(w, npx, cxs)
        cys = jnp.where(w, npy, cys)
        czs = jnp.where(w, npz, czs)
        return (npx, npy, npz, cxs, cys, czs, nmask)

    carry = (px, py, pz, cxs, cys, czs, mask)
    _, _, _, cxs, cys, czs, _ = lax.fori_loop(0, _NP - 1, body, carry)
    cx_ref[...] = cxs
    cy_ref[...] = cys
    cz_ref[...] = czs


def _ballq_body(xyz_ref, cx_ref, cy_ref, cz_ref, out_ref, key_scr):
    b = pl.program_id(0)
    j = pl.program_id(1)
    c0 = pl.multiple_of(j * _CB, 128)
    x = xyz_ref[0, 0, :]
    y = xyz_ref[0, 1, :]
    z = xyz_ref[0, 2, :]
    cx = cx_ref[0, 0, pl.ds(c0, _CB)]
    cy = cy_ref[0, 0, pl.ds(c0, _CB)]
    cz = cz_ref[0, 0, pl.ds(c0, _CB)]
    dx = x[None, :] - cx[:, None]
    dy = y[None, :] - cy[:, None]
    dz = z[None, :] - cz[:, None]
    d = jnp.sqrt(dx * dx + dy * dy + dz * dz)
    dc = jnp.minimum(d, _T04)
    iota = lax.broadcasted_iota(jnp.int32, (_CB, _N), 1)
    key = jnp.where(dc < _T04, dc, jnp.float32(1.0) + iota.astype(jnp.float32))
    key_scr[...] = key
    base = b * _N
    krow = lax.broadcasted_iota(jnp.int32, (_NS, _CB), 0)

    def body(k, acc):
        key = key_scr[...]
        mn = jnp.min(key, axis=1, keepdims=True)
        idx = jnp.min(jnp.where(key == mn, iota, jnp.int32(_N)), axis=1)
        acc = jnp.where(krow == k, (idx + base)[None, :], acc)
        key_scr[...] = jnp.where(iota == idx[:, None], jnp.float32(jnp.inf),
                                 key)
        return acc

    acc = lax.fori_loop(0, _NS, body, jnp.zeros((_NS, _CB), jnp.int32))
    out_ref[0, :, pl.ds(c0, _CB)] = acc


def _mm_body(t_ref, w_ref, out_ref):
    out_ref[...] = lax.dot_general(
        t_ref[...], w_ref[...], (((1,), (0,)), ((), ())),
        precision=lax.Precision.HIGHEST, preferred_element_type=jnp.float32)


def _mlp_body(x_ref, b1_ref, w2_ref, b2_ref, w3_ref, b3_ref, out_ref):
    dn = (((1,), (0,)), ((), ()))
    h = jnp.maximum(x_ref[...] + b1_ref[...], jnp.float32(0.0))
    h = lax.dot_general(h, w2_ref[...], dn, precision=lax.Precision.HIGHEST,
                        preferred_element_type=jnp.float32)
    h = jnp.maximum(h + b2_ref[...], jnp.float32(0.0))
    h = lax.dot_general(h, w3_ref[...], dn, precision=lax.Precision.HIGHEST,
                        preferred_element_type=jnp.float32)
    h = jnp.maximum(h + b3_ref[...], jnp.float32(0.0))
    out_ref[...] = jnp.max(h.reshape(_NS, _NS, 128), axis=1)


def _sc_gather(g, idx):
    info = plsc.get_sparse_core_info()
    ncores = info.num_cores
    nw = ncores * info.num_subcores
    rows = idx.shape[0]
    per_w = rows // nw
    chunk = 128
    nchunks = per_w // chunk
    mesh = plsc.VectorSubcoreMesh(core_axis_name="c", subcore_axis_name="s")

    @functools.partial(
        pl.kernel, mesh=mesh,
        out_type=jax.ShapeDtypeStruct((rows, g.shape[1]), jnp.float32),
        scratch_types=[
            pltpu.VMEM((chunk,), jnp.int32),
            pltpu.VMEM((chunk, g.shape[1]), jnp.float32),
            pltpu.SemaphoreType.DMA,
        ],
    )
    def k(g_hbm, idx_hbm, out_hbm, idx_v, rows_v, sem):
        wid = lax.axis_index("s") * ncores + lax.axis_index("c")

        def body(c, carry):
            b0 = wid * per_w + c * chunk
            pltpu.sync_copy(idx_hbm.at[pl.ds(b0, chunk)], idx_v)
            pltpu.async_copy(g_hbm.at[idx_v], rows_v, sem).wait()
            pltpu.sync_copy(rows_v, out_hbm.at[pl.ds(b0, chunk)])
            return carry

        lax.fori_loop(0, nchunks, body, 0)

    return k(g, idx)


def kernel(xyz, points, W1, b1, W2, b2, W3, b3):
    B, N, _ = xyz.shape
    f32 = jnp.float32
    init_id = jax.random.randint(jax.random.key(42), (B,), 0, N - 1)
    init_f = init_id.astype(f32).reshape(B, 1)
    xyz_t = jnp.transpose(xyz, (0, 2, 1))

    cshape = jax.ShapeDtypeStruct((B, _NP), f32)
    cx, cy, cz = pl.pallas_call(
        _fps_body,
        out_shape=[cshape, cshape, cshape],
    )(xyz_t, init_f)

    gidx = pl.pallas_call(
        _ballq_body,
        grid=(B, _NP // _CB),
        in_specs=[
            pl.BlockSpec((1, 3, N), lambda b, j: (b, 0, 0)),
            pl.BlockSpec((1, 1, _NP), lambda b, j: (b, 0, 0)),
            pl.BlockSpec((1, 1, _NP), lambda b, j: (b, 0, 0)),
            pl.BlockSpec((1, 1, _NP), lambda b, j: (b, 0, 0)),
        ],
        out_specs=pl.BlockSpec((1, _NS, _NP), lambda b, j: (b, 0, 0)),
        out_shape=jax.ShapeDtypeStruct((B, _NS, _NP), jnp.int32),
        scratch_shapes=[pltpu.VMEM((_CB, _N), f32)],
    )(xyz_t, cx.reshape(B, 1, _NP), cy.reshape(B, 1, _NP),
      cz.reshape(B, 1, _NP))
    gidx_flat = gidx.transpose(0, 2, 1).reshape(-1)

    W1p = jnp.pad(W1, ((0, 0), (0, 64)))
    b1p = jnp.pad(b1, (0, 64))
    W2p = jnp.pad(W2, ((0, 64), (0, 0)))

    T = jnp.concatenate([xyz, points], axis=-1).reshape(B * N, 3 + _DP)
    RB = 2048
    G = pl.pallas_call(
        _mm_body,
        grid=(B * N // RB,),
        in_specs=[
            pl.BlockSpec((RB, 3 + _DP), lambda i: (i, 0)),
            pl.BlockSpec((3 + _DP, 128), lambda i: (0, 0)),
        ],
        out_specs=pl.BlockSpec((RB, 128), lambda i: (i, 0)),
        out_shape=jax.ShapeDtypeStruct((B * N, 128), f32),
    )(T, W1p)

    X1 = _sc_gather(G, gidx_flat)

    MB = _NS * _NS
    out = pl.pallas_call(
        _mlp_body,
        grid=(B * _NP * _NS // MB,),
        in_specs=[
            pl.BlockSpec((MB, 128), lambda i: (i, 0)),
            pl.BlockSpec((1, 128), lambda i: (0, 0)),
            pl.BlockSpec((128, 64), lambda i: (0, 0)),
            pl.BlockSpec((1, 64), lambda i: (0, 0)),
            pl.BlockSpec((64, 128), lambda i: (0, 0)),
            pl.BlockSpec((1, 128), lambda i: (0, 0)),
        ],
        out_specs=pl.BlockSpec((_NS, 128), lambda i: (i, 0)),
        out_shape=jax.ShapeDtypeStruct((B * _NP, 128), f32),
    )(X1, b1p.reshape(1, 128), W2p, b2.reshape(1, 64), W3, b3.reshape(1, 128))

    cent_xyz = jnp.stack([cx, cy, cz], axis=-1)
    return (cent_xyz, out.reshape(B, _NP, 128))


# speedup vs baseline: 12.9678x; 1.2126x over previous
import functools

import numpy as np
import jax
import jax.numpy as jnp
from jax import lax
from jax.experimental import pallas as pl
from jax.experimental.pallas import tpu as pltpu
from jax.experimental.pallas import tpu_sc as plsc

_B, _N, _DP = 8, 4096, 64
_NP = 512
_NS = 32
_CB = 128
_T04 = np.float32(0.2 ** 2)


def _fps_body(xyz_ref, init_ref, cx_ref, cy_ref, cz_ref):
    X = xyz_ref[:, 0, :]
    Y = xyz_ref[:, 1, :]
    Z = xyz_ref[:, 2, :]
    iota = lax.broadcasted_iota(jnp.int32, (_B, _N), 1)
    slot = lax.broadcasted_iota(jnp.int32, (_B, _NP), 1)
    zero = jnp.zeros((_B, _N), jnp.float32)
    zc = jnp.zeros((_B, _NP), jnp.float32)

    def pick(sel):
        px = jnp.sum(jnp.where(sel, X, zero), axis=1, keepdims=True)
        py = jnp.sum(jnp.where(sel, Y, zero), axis=1, keepdims=True)
        pz = jnp.sum(jnp.where(sel, Z, zero), axis=1, keepdims=True)
        return px, py, pz

    init_i = init_ref[...].astype(jnp.int32)
    px, py, pz = pick(iota == init_i)
    cxs = jnp.where(slot == 0, px, zc)
    cys = jnp.where(slot == 0, py, zc)
    czs = jnp.where(slot == 0, pz, zc)
    mask = jnp.ones((_B, _N), jnp.float32)

    def body(i, carry):
        px, py, pz, cxs, cys, czs, mask = carry
        dx = X - px
        dy = Y - py
        dz = Z - pz
        d = jnp.sqrt(dx * dx + dy * dy + dz * dz)
        dm = d * mask
        mx = jnp.max(dm, axis=1, keepdims=True)
        idx = jnp.min(jnp.where(dm == mx, iota, jnp.int32(_N)), axis=1,
                      keepdims=True)
        npx, npy, npz = pick(iota == idx)
        nmask = jnp.minimum(dm * mask * jnp.float32(1e11), mask)
        w = slot == (i + 1)
        cxs = jnp.where(w, npx, cxs)
        cys = jnp.where(w, npy, cys)
        czs = jnp.where(w, npz, czs)
        return (npx, npy, npz, cxs, cys, czs, nmask)

    carry = (px, py, pz, cxs, cys, czs, mask)
    _, _, _, cxs, cys, czs, _ = lax.fori_loop(0, _NP - 1, body, carry)
    cx_ref[...] = cxs
    cy_ref[...] = cys
    cz_ref[...] = czs


def _ballq_body(xyz_ref, cx_ref, cy_ref, cz_ref, out_ref, key_scr):
    b = pl.program_id(0)
    j = pl.program_id(1)
    c0 = pl.multiple_of(j * _CB, 128)
    x = xyz_ref[0, 0, :]
    y = xyz_ref[0, 1, :]
    z = xyz_ref[0, 2, :]
    cx = cx_ref[0, 0, pl.ds(c0, _CB)]
    cy = cy_ref[0, 0, pl.ds(c0, _CB)]
    cz = cz_ref[0, 0, pl.ds(c0, _CB)]
    dx = x[None, :] - cx[:, None]
    dy = y[None, :] - cy[:, None]
    dz = z[None, :] - cz[:, None]
    d = jnp.sqrt(dx * dx + dy * dy + dz * dz)
    dc = jnp.minimum(d, _T04)
    iota = lax.broadcasted_iota(jnp.int32, (_CB, _N), 1)
    key = jnp.where(dc < _T04, dc, jnp.float32(1.0) + iota.astype(jnp.float32))
    key_scr[...] = key
    base = b * _N
    krow = lax.broadcasted_iota(jnp.int32, (_NS, _CB), 0)
    one = jnp.float32(1.0)
    inf = jnp.float32(jnp.inf)

    def p1_cond(carry):
        _, _, mn = carry
        return jnp.min(mn) < one

    def p1_body(carry):
        acc, cnt, mn = carry
        key = key_scr[...]
        idx = jnp.min(jnp.where(key == mn[:, None], iota, jnp.int32(_N)),
                      axis=1)
        act = mn < one
        acc = jnp.where((krow == cnt[None, :]) & act[None, :],
                        (idx + base)[None, :], acc)
        cnt = cnt + act.astype(jnp.int32)
        key = jnp.where((iota == idx[:, None]) & act[:, None], inf, key)
        key_scr[...] = key
        return (acc, cnt, jnp.min(key, axis=1))

    acc0 = jnp.zeros((_NS, _CB), jnp.int32)
    cnt0 = jnp.zeros((_CB,), jnp.int32)
    mn0 = jnp.min(key, axis=1)
    acc, cnt, _ = lax.while_loop(p1_cond, p1_body, (acc0, cnt0, mn0))

    iota64 = lax.broadcasted_iota(jnp.int32, (_CB, 64), 1)

    def p2_body(k, carry):
        acc, cnt = carry
        sub = key_scr[:, :64]
        mn = jnp.min(sub, axis=1)
        idx = jnp.min(jnp.where(sub == mn[:, None], iota64, jnp.int32(_N)),
                      axis=1)
        acc = jnp.where(krow == cnt[None, :], (idx + base)[None, :], acc)
        cnt = cnt + 1
        key_scr[:, :64] = jnp.where(iota64 == idx[:, None], inf, sub)
        return (acc, cnt)

    acc, _ = lax.fori_loop(0, _NS, p2_body, (acc, cnt))
    out_ref[0, :, pl.ds(c0, _CB)] = acc


def _mm_body(t_ref, w_ref, out_ref):
    out_ref[...] = lax.dot_general(
        t_ref[...], w_ref[...], (((1,), (0,)), ((), ())),
        precision=lax.Precision.HIGHEST, preferred_element_type=jnp.float32)


def _mlp_body(x_ref, b1_ref, w2_ref, b2_ref, w3_ref, b3_ref, out_ref):
    dn = (((1,), (0,)), ((), ()))
    h = jnp.maximum(x_ref[...] + b1_ref[...], jnp.float32(0.0))
    h = lax.dot_general(h, w2_ref[...], dn, precision=lax.Precision.HIGHEST,
                        preferred_element_type=jnp.float32)
    h = jnp.maximum(h + b2_ref[...], jnp.float32(0.0))
    h = lax.dot_general(h, w3_ref[...], dn, precision=lax.Precision.HIGHEST,
                        preferred_element_type=jnp.float32)
    h = jnp.maximum(h + b3_ref[...], jnp.float32(0.0))
    out_ref[...] = jnp.max(h.reshape(_NS, _NS, 128), axis=1)


def _sc_gather(g, idx):
    info = plsc.get_sparse_core_info()
    ncores = info.num_cores
    nw = ncores * info.num_subcores
    rows = idx.shape[0]
    per_w = rows // nw
    chunk = 128
    nchunks = per_w // chunk
    mesh = plsc.VectorSubcoreMesh(core_axis_name="c", subcore_axis_name="s")

    @functools.partial(
        pl.kernel, mesh=mesh,
        out_type=jax.ShapeDtypeStruct((rows, g.shape[1]), jnp.float32),
        scratch_types=[
            pltpu.VMEM((chunk,), jnp.int32),
            pltpu.VMEM((chunk, g.shape[1]), jnp.float32),
            pltpu.SemaphoreType.DMA,
        ],
    )
    def k(g_hbm, idx_hbm, out_hbm, idx_v, rows_v, sem):
        wid = lax.axis_index("s") * ncores + lax.axis_index("c")

        def body(c, carry):
            b0 = wid * per_w + c * chunk
            pltpu.sync_copy(idx_hbm.at[pl.ds(b0, chunk)], idx_v)
            pltpu.async_copy(g_hbm.at[idx_v], rows_v, sem).wait()
            pltpu.sync_copy(rows_v, out_hbm.at[pl.ds(b0, chunk)])
            return carry

        lax.fori_loop(0, nchunks, body, 0)

    return k(g, idx)


def kernel(xyz, points, W1, b1, W2, b2, W3, b3):
    B, N, _ = xyz.shape
    f32 = jnp.float32
    init_id = jax.random.randint(jax.random.key(42), (B,), 0, N - 1)
    init_f = init_id.astype(f32).reshape(B, 1)
    xyz_t = jnp.transpose(xyz, (0, 2, 1))

    cshape = jax.ShapeDtypeStruct((B, _NP), f32)
    cx, cy, cz = pl.pallas_call(
        _fps_body,
        out_shape=[cshape, cshape, cshape],
    )(xyz_t, init_f)

    gidx = pl.pallas_call(
        _ballq_body,
        grid=(B, _NP // _CB),
        in_specs=[
            pl.BlockSpec((1, 3, N), lambda b, j: (b, 0, 0)),
            pl.BlockSpec((1, 1, _NP), lambda b, j: (b, 0, 0)),
            pl.BlockSpec((1, 1, _NP), lambda b, j: (b, 0, 0)),
            pl.BlockSpec((1, 1, _NP), lambda b, j: (b, 0, 0)),
        ],
        out_specs=pl.BlockSpec((1, _NS, _NP), lambda b, j: (b, 0, 0)),
        out_shape=jax.ShapeDtypeStruct((B, _NS, _NP), jnp.int32),
        scratch_shapes=[pltpu.VMEM((_CB, _N), f32)],
    )(xyz_t, cx.reshape(B, 1, _NP), cy.reshape(B, 1, _NP),
      cz.reshape(B, 1, _NP))
    gidx_flat = gidx.transpose(0, 2, 1).reshape(-1)

    W1p = jnp.pad(W1, ((0, 0), (0, 64)))
    b1p = jnp.pad(b1, (0, 64))
    W2p = jnp.pad(W2, ((0, 64), (0, 0)))

    T = jnp.concatenate([xyz, points], axis=-1).reshape(B * N, 3 + _DP)
    RB = 2048
    G = pl.pallas_call(
        _mm_body,
        grid=(B * N // RB,),
        in_specs=[
            pl.BlockSpec((RB, 3 + _DP), lambda i: (i, 0)),
            pl.BlockSpec((3 + _DP, 128), lambda i: (0, 0)),
        ],
        out_specs=pl.BlockSpec((RB, 128), lambda i: (i, 0)),
        out_shape=jax.ShapeDtypeStruct((B * N, 128), f32),
    )(T, W1p)

    X1 = _sc_gather(G, gidx_flat)

    MB = _NS * _NS
    out = pl.pallas_call(
        _mlp_body,
        grid=(B * _NP * _NS // MB,),
        in_specs=[
            pl.BlockSpec((MB, 128), lambda i: (i, 0)),
            pl.BlockSpec((1, 128), lambda i: (0, 0)),
            pl.BlockSpec((128, 64), lambda i: (0, 0)),
            pl.BlockSpec((1, 64), lambda i: (0, 0)),
            pl.BlockSpec((64, 128), lambda i: (0, 0)),
            pl.BlockSpec((1, 128), lambda i: (0, 0)),
        ],
        out_specs=pl.BlockSpec((_NS, 128), lambda i: (i, 0)),
        out_shape=jax.ShapeDtypeStruct((B * _NP, 128), f32),
    )(X1, b1p.reshape(1, 128), W2p, b2.reshape(1, 64), W3, b3.reshape(1, 128))

    cent_xyz = jnp.stack([cx, cy, cz], axis=-1)
    return (cent_xyz, out.reshape(B, _NP, 128))


# speedup vs baseline: 12.9696x; 1.0001x over previous
import functools

import numpy as np
import jax
import jax.numpy as jnp
from jax import lax
from jax.experimental import pallas as pl
from jax.experimental.pallas import tpu as pltpu
from jax.experimental.pallas import tpu_sc as plsc

_B, _N, _DP = 8, 4096, 64
_NP = 512
_NS = 32
_CB = 128
_T04 = np.float32(0.2 ** 2)


def _fps_body(xyz_ref, init_ref, cx_ref, cy_ref, cz_ref):
    X = xyz_ref[:, 0, :]
    Y = xyz_ref[:, 1, :]
    Z = xyz_ref[:, 2, :]
    iota = lax.broadcasted_iota(jnp.int32, (_B, _N), 1)
    slot = lax.broadcasted_iota(jnp.int32, (_B, _NP), 1)
    zero = jnp.zeros((_B, _N), jnp.float32)
    zc = jnp.zeros((_B, _NP), jnp.float32)

    def pick(sel):
        px = jnp.sum(jnp.where(sel, X, zero), axis=1, keepdims=True)
        py = jnp.sum(jnp.where(sel, Y, zero), axis=1, keepdims=True)
        pz = jnp.sum(jnp.where(sel, Z, zero), axis=1, keepdims=True)
        return px, py, pz

    init_i = init_ref[...].astype(jnp.int32)
    px, py, pz = pick(iota == init_i)
    cxs = jnp.where(slot == 0, px, zc)
    cys = jnp.where(slot == 0, py, zc)
    czs = jnp.where(slot == 0, pz, zc)
    mask = jnp.ones((_B, _N), jnp.float32)

    def body(i, carry):
        px, py, pz, cxs, cys, czs, mask = carry
        dx = X - px
        dy = Y - py
        dz = Z - pz
        d = jnp.sqrt(dx * dx + dy * dy + dz * dz)
        dm = d * mask
        mx = jnp.max(dm, axis=1, keepdims=True)
        idx = jnp.min(jnp.where(dm == mx, iota, jnp.int32(_N)), axis=1,
                      keepdims=True)
        npx, npy, npz = pick(iota == idx)
        nmask = jnp.minimum(dm * mask * jnp.float32(1e11), mask)
        w = slot == (i + 1)
        cxs = jnp.where(w, npx, cxs)
        cys = jnp.where(w, npy, cys)
        czs = jnp.where(w, npz, czs)
        return (npx, npy, npz, cxs, cys, czs, nmask)

    carry = (px, py, pz, cxs, cys, czs, mask)
    _, _, _, cxs, cys, czs, _ = lax.fori_loop(0, _NP - 1, body, carry)
    cx_ref[...] = cxs
    cy_ref[...] = cys
    cz_ref[...] = czs


def _ballq_body(xyz_ref, cx_ref, cy_ref, cz_ref, out_ref, key_scr):
    b = pl.program_id(0)
    j = pl.program_id(1)
    c0 = pl.multiple_of(j * _CB, 128)
    x = xyz_ref[0, 0, :]
    y = xyz_ref[0, 1, :]
    z = xyz_ref[0, 2, :]
    cx = cx_ref[0, 0, pl.ds(c0, _CB)]
    cy = cy_ref[0, 0, pl.ds(c0, _CB)]
    cz = cz_ref[0, 0, pl.ds(c0, _CB)]
    dx = x[None, :] - cx[:, None]
    dy = y[None, :] - cy[:, None]
    dz = z[None, :] - cz[:, None]
    d = jnp.sqrt(dx * dx + dy * dy + dz * dz)
    dc = jnp.minimum(d, _T04)
    iota = lax.broadcasted_iota(jnp.int32, (_CB, _N), 1)
    key = jnp.where(dc < _T04, dc, jnp.float32(1.0) + iota.astype(jnp.float32))
    key_scr[...] = key
    base = b * _N
    krow = lax.broadcasted_iota(jnp.int32, (_NS, _CB), 0)
    one = jnp.float32(1.0)
    inf = jnp.float32(jnp.inf)

    def p1_cond(carry):
        _, _, mn = carry
        return jnp.min(mn) < one

    def p1_body(carry):
        acc, cnt, mn = carry
        key = key_scr[...]
        idx = jnp.min(jnp.where(key == mn[:, None], iota, jnp.int32(_N)),
                      axis=1)
        act = mn < one
        acc = jnp.where((krow == cnt[None, :]) & act[None, :],
                        (idx + base)[None, :], acc)
        cnt = cnt + act.astype(jnp.int32)
        key = jnp.where((iota == idx[:, None]) & act[:, None], inf, key)
        key_scr[...] = key
        return (acc, cnt, jnp.min(key, axis=1))

    acc0 = jnp.zeros((_NS, _CB), jnp.int32)
    cnt0 = jnp.zeros((_CB,), jnp.int32)
    mn0 = jnp.min(key, axis=1)
    acc, cnt, _ = lax.while_loop(p1_cond, p1_body, (acc0, cnt0, mn0))

    iota64 = lax.broadcasted_iota(jnp.int32, (_CB, 64), 1)

    def p2_body(k, carry):
        acc, cnt = carry
        sub = key_scr[:, :64]
        mn = jnp.min(sub, axis=1)
        idx = jnp.min(jnp.where(sub == mn[:, None], iota64, jnp.int32(_N)),
                      axis=1)
        acc = jnp.where(krow == cnt[None, :], (idx + base)[None, :], acc)
        cnt = cnt + 1
        key_scr[:, :64] = jnp.where(iota64 == idx[:, None], inf, sub)
        return (acc, cnt)

    acc, _ = lax.fori_loop(0, _NS, p2_body, (acc, cnt))
    out_ref[0, :, pl.ds(c0, _CB)] = acc


def _mm_body(t_ref, w_ref, out_ref):
    out_ref[...] = lax.dot_general(
        t_ref[...], w_ref[...], (((1,), (0,)), ((), ())),
        precision=lax.Precision.HIGHEST, preferred_element_type=jnp.float32)


def _mlp_body(x_ref, b1_ref, w2_ref, b2_ref, w3_ref, b3_ref, out_ref):
    dn = (((1,), (0,)), ((), ()))
    h = jnp.maximum(x_ref[...] + b1_ref[...], jnp.float32(0.0))
    h = lax.dot_general(h, w2_ref[...], dn, precision=lax.Precision.HIGHEST,
                        preferred_element_type=jnp.float32)
    h = jnp.maximum(h + b2_ref[...], jnp.float32(0.0))
    h = lax.dot_general(h, w3_ref[...], dn, precision=lax.Precision.HIGHEST,
                        preferred_element_type=jnp.float32)
    h = jnp.maximum(h + b3_ref[...], jnp.float32(0.0))
    out_ref[...] = jnp.max(h.reshape(_NS, _NS, 128), axis=1)


def _sc_gather(g, idx):
    info = plsc.get_sparse_core_info()
    ncores = info.num_cores
    nw = ncores * info.num_subcores
    rows = idx.shape[0]
    per_w = rows // nw
    chunk = 128
    nchunks = per_w // chunk
    mesh = plsc.VectorSubcoreMesh(core_axis_name="c", subcore_axis_name="s")

    @functools.partial(
        pl.kernel, mesh=mesh,
        out_type=jax.ShapeDtypeStruct((rows, g.shape[1]), jnp.float32),
        scratch_types=[
            pltpu.VMEM((chunk,), jnp.int32),
            pltpu.VMEM((chunk,), jnp.int32),
            pltpu.VMEM((chunk, g.shape[1]), jnp.float32),
            pltpu.VMEM((chunk, g.shape[1]), jnp.float32),
            pltpu.SemaphoreType.DMA,
            pltpu.SemaphoreType.DMA,
        ],
    )
    def k(g_hbm, idx_hbm, out_hbm, iv0, iv1, rv0, rv1, sg0, sg1):
        wid = lax.axis_index("s") * ncores + lax.axis_index("c")
        base = wid * per_w
        iv = (iv0, iv1)
        rv = (rv0, rv1)
        sg = (sg0, sg1)

        pltpu.sync_copy(idx_hbm.at[pl.ds(base, chunk)], iv0)
        copies = [pltpu.async_copy(g_hbm.at[iv0], rv0, sg0), None]
        pltpu.sync_copy(idx_hbm.at[pl.ds(base + chunk, chunk)], iv1)
        for c in range(nchunks):
            cur, nxt = c % 2, (c + 1) % 2
            copies[cur].wait()
            if c + 1 < nchunks:
                copies[nxt] = pltpu.async_copy(g_hbm.at[iv[nxt]], rv[nxt],
                                               sg[nxt])
            pltpu.sync_copy(rv[cur], out_hbm.at[pl.ds(base + c * chunk,
                                                      chunk)])
            if c + 2 < nchunks:
                pltpu.sync_copy(idx_hbm.at[pl.ds(base + (c + 2) * chunk,
                                                 chunk)], iv[cur])

    return k(g, idx)


def kernel(xyz, points, W1, b1, W2, b2, W3, b3):
    B, N, _ = xyz.shape
    f32 = jnp.float32
    init_id = jax.random.randint(jax.random.key(42), (B,), 0, N - 1)
    init_f = init_id.astype(f32).reshape(B, 1)
    xyz_t = jnp.transpose(xyz, (0, 2, 1))

    cshape = jax.ShapeDtypeStruct((B, _NP), f32)
    cx, cy, cz = pl.pallas_call(
        _fps_body,
        out_shape=[cshape, cshape, cshape],
    )(xyz_t, init_f)

    gidx = pl.pallas_call(
        _ballq_body,
        grid=(B, _NP // _CB),
        in_specs=[
            pl.BlockSpec((1, 3, N), lambda b, j: (b, 0, 0)),
            pl.BlockSpec((1, 1, _NP), lambda b, j: (b, 0, 0)),
            pl.BlockSpec((1, 1, _NP), lambda b, j: (b, 0, 0)),
            pl.BlockSpec((1, 1, _NP), lambda b, j: (b, 0, 0)),
        ],
        out_specs=pl.BlockSpec((1, _NS, _NP), lambda b, j: (b, 0, 0)),
        out_shape=jax.ShapeDtypeStruct((B, _NS, _NP), jnp.int32),
        scratch_shapes=[pltpu.VMEM((_CB, _N), f32)],
    )(xyz_t, cx.reshape(B, 1, _NP), cy.reshape(B, 1, _NP),
      cz.reshape(B, 1, _NP))
    gidx_flat = gidx.transpose(0, 2, 1).reshape(-1)

    W1p = jnp.pad(W1, ((0, 0), (0, 64)))
    b1p = jnp.pad(b1, (0, 64))
    W2p = jnp.pad(W2, ((0, 64), (0, 0)))

    T = jnp.concatenate([xyz, points], axis=-1).reshape(B * N, 3 + _DP)
    RB = 2048
    G = pl.pallas_call(
        _mm_body,
        grid=(B * N // RB,),
        in_specs=[
            pl.BlockSpec((RB, 3 + _DP), lambda i: (i, 0)),
            pl.BlockSpec((3 + _DP, 128), lambda i: (0, 0)),
        ],
        out_specs=pl.BlockSpec((RB, 128), lambda i: (i, 0)),
        out_shape=jax.ShapeDtypeStruct((B * N, 128), f32),
    )(T, W1p)

    X1 = _sc_gather(G, gidx_flat)

    MB = _NS * _NS
    out = pl.pallas_call(
        _mlp_body,
        grid=(B * _NP * _NS // MB,),
        in_specs=[
            pl.BlockSpec((MB, 128), lambda i: (i, 0)),
            pl.BlockSpec((1, 128), lambda i: (0, 0)),
            pl.BlockSpec((128, 64), lambda i: (0, 0)),
            pl.BlockSpec((1, 64), lambda i: (0, 0)),
            pl.BlockSpec((64, 128), lambda i: (0, 0)),
            pl.BlockSpec((1, 128), lambda i: (0, 0)),
        ],
        out_specs=pl.BlockSpec((_NS, 128), lambda i: (i, 0)),
        out_shape=jax.ShapeDtypeStruct((B * _NP, 128), f32),
    )(X1, b1p.reshape(1, 128), W2p, b2.reshape(1, 64), W3, b3.reshape(1, 128))

    cent_xyz = jnp.stack([cx, cy, cz], axis=-1)
    return (cent_xyz, out.reshape(B, _NP, 128))


# speedup vs baseline: 14.7097x; 1.1342x over previous
import functools

import numpy as np
import jax
import jax.numpy as jnp
from jax import lax
from jax.experimental import pallas as pl
from jax.experimental.pallas import tpu as pltpu
from jax.experimental.pallas import tpu_sc as plsc

_B, _N, _DP = 8, 4096, 64
_NP = 512
_NS = 32
_CB = 128
_T04 = np.float32(0.2 ** 2)


def _fps_body(xyz_ref, init_ref, cx_ref, cy_ref, cz_ref):
    X = xyz_ref[:, 0, :]
    Y = xyz_ref[:, 1, :]
    Z = xyz_ref[:, 2, :]
    iota = lax.broadcasted_iota(jnp.int32, (_B, _N), 1)
    slot = lax.broadcasted_iota(jnp.int32, (_B, _NP), 1)
    zero = jnp.zeros((_B, _N), jnp.float32)
    zc = jnp.zeros((_B, _NP), jnp.float32)

    def pick(sel):
        px = jnp.sum(jnp.where(sel, X, zero), axis=1, keepdims=True)
        py = jnp.sum(jnp.where(sel, Y, zero), axis=1, keepdims=True)
        pz = jnp.sum(jnp.where(sel, Z, zero), axis=1, keepdims=True)
        return px, py, pz

    init_i = init_ref[...].astype(jnp.int32)
    px, py, pz = pick(iota == init_i)
    cxs = jnp.where(slot == 0, px, zc)
    cys = jnp.where(slot == 0, py, zc)
    czs = jnp.where(slot == 0, pz, zc)
    mask = jnp.ones((_B, _N), jnp.float32)

    def body(i, carry):
        px, py, pz, cxs, cys, czs, mask = carry
        dx = X - px
        dy = Y - py
        dz = Z - pz
        d = jnp.sqrt(dx * dx + dy * dy + dz * dz)
        dm = d * mask
        mx = jnp.max(dm, axis=1, keepdims=True)
        idx = jnp.min(jnp.where(dm == mx, iota, jnp.int32(_N)), axis=1,
                      keepdims=True)
        npx, npy, npz = pick(iota == idx)
        nmask = jnp.minimum(dm * mask * jnp.float32(1e11), mask)
        w = slot == (i + 1)
        cxs = jnp.where(w, npx, cxs)
        cys = jnp.where(w, npy, cys)
        czs = jnp.where(w, npz, czs)
        return (npx, npy, npz, cxs, cys, czs, nmask)

    carry = (px, py, pz, cxs, cys, czs, mask)
    _, _, _, cxs, cys, czs, _ = lax.fori_loop(0, _NP - 1, body, carry)
    cx_ref[...] = cxs
    cy_ref[...] = cys
    cz_ref[...] = czs


def _ballq_body(xyz_ref, cx_ref, cy_ref, cz_ref, out_ref, key_scr):
    b = pl.program_id(0)
    j = pl.program_id(1)
    c0 = pl.multiple_of(j * _CB, 128)
    x = xyz_ref[0, 0, :]
    y = xyz_ref[0, 1, :]
    z = xyz_ref[0, 2, :]
    cx = cx_ref[0, 0, pl.ds(c0, _CB)]
    cy = cy_ref[0, 0, pl.ds(c0, _CB)]
    cz = cz_ref[0, 0, pl.ds(c0, _CB)]
    dx = x[None, :] - cx[:, None]
    dy = y[None, :] - cy[:, None]
    dz = z[None, :] - cz[:, None]
    d = jnp.sqrt(dx * dx + dy * dy + dz * dz)
    dc = jnp.minimum(d, _T04)
    iota = lax.broadcasted_iota(jnp.int32, (_CB, _N), 1)
    key = jnp.where(dc < _T04, dc, jnp.float32(1.0) + iota.astype(jnp.float32))
    key_scr[...] = key
    base = b * _N
    krow = lax.broadcasted_iota(jnp.int32, (_NS, _CB), 0)
    one = jnp.float32(1.0)
    inf = jnp.float32(jnp.inf)

    def p1_cond(carry):
        _, _, mn = carry
        return jnp.min(mn) < one

    def p1_body(carry):
        acc, cnt, mn = carry
        key = key_scr[...]
        idx = jnp.min(jnp.where(key == mn[:, None], iota, jnp.int32(_N)),
                      axis=1)
        act = mn < one
        acc = jnp.where((krow == cnt[None, :]) & act[None, :],
                        (idx + base)[None, :], acc)
        cnt = cnt + act.astype(jnp.int32)
        key = jnp.where((iota == idx[:, None]) & act[:, None], inf, key)
        key_scr[...] = key
        return (acc, cnt, jnp.min(key, axis=1))

    acc0 = jnp.zeros((_NS, _CB), jnp.int32)
    cnt0 = jnp.zeros((_CB,), jnp.int32)
    mn0 = jnp.min(key, axis=1)
    acc, cnt, _ = lax.while_loop(p1_cond, p1_body, (acc0, cnt0, mn0))

    iota64 = lax.broadcasted_iota(jnp.int32, (_CB, 64), 1)

    def p2_body(k, carry):
        acc, cnt = carry
        sub = key_scr[:, :64]
        mn = jnp.min(sub, axis=1)
        idx = jnp.min(jnp.where(sub == mn[:, None], iota64, jnp.int32(_N)),
                      axis=1)
        acc = jnp.where(krow == cnt[None, :], (idx + base)[None, :], acc)
        cnt = cnt + 1
        key_scr[:, :64] = jnp.where(iota64 == idx[:, None], inf, sub)
        return (acc, cnt)

    acc, _ = lax.fori_loop(0, _NS, p2_body, (acc, cnt))
    out_ref[0, :, pl.ds(c0, _CB)] = acc


def _mm_body(t_ref, w_ref, out_ref):
    out_ref[...] = lax.dot_general(
        t_ref[...], w_ref[...], (((1,), (0,)), ((), ())),
        preferred_element_type=jnp.float32)


def _mlp_body(x_ref, b1_ref, w2_ref, b2_ref, w3_ref, b3_ref, out_ref):
    dn = (((1,), (0,)), ((), ()))
    h = jnp.maximum(x_ref[...] + b1_ref[...], jnp.float32(0.0))
    h = lax.dot_general(h, w2_ref[...], dn,
                        preferred_element_type=jnp.float32)
    h = jnp.maximum(h + b2_ref[...], jnp.float32(0.0))
    h = lax.dot_general(h, w3_ref[...], dn,
                        preferred_element_type=jnp.float32)
    h = jnp.maximum(h + b3_ref[...], jnp.float32(0.0))
    out_ref[...] = jnp.max(h.reshape(_NS, _NS, 128), axis=1)


def _sc_gather(g, idx):
    info = plsc.get_sparse_core_info()
    ncores = info.num_cores
    nw = ncores * info.num_subcores
    rows = idx.shape[0]
    per_w = rows // nw
    chunk = 128
    nchunks = per_w // chunk
    mesh = plsc.VectorSubcoreMesh(core_axis_name="c", subcore_axis_name="s")

    @functools.partial(
        pl.kernel, mesh=mesh,
        out_type=jax.ShapeDtypeStruct((rows, g.shape[1]), jnp.float32),
        scratch_types=[
            pltpu.VMEM((chunk,), jnp.int32),
            pltpu.VMEM((chunk,), jnp.int32),
            pltpu.VMEM((chunk, g.shape[1]), jnp.float32),
            pltpu.VMEM((chunk, g.shape[1]), jnp.float32),
            pltpu.SemaphoreType.DMA,
            pltpu.SemaphoreType.DMA,
        ],
    )
    def k(g_hbm, idx_hbm, out_hbm, iv0, iv1, rv0, rv1, sg0, sg1):
        wid = lax.axis_index("s") * ncores + lax.axis_index("c")
        base = wid * per_w
        iv = (iv0, iv1)
        rv = (rv0, rv1)
        sg = (sg0, sg1)

        pltpu.sync_copy(idx_hbm.at[pl.ds(base, chunk)], iv0)
        copies = [pltpu.async_copy(g_hbm.at[iv0], rv0, sg0), None]
        pltpu.sync_copy(idx_hbm.at[pl.ds(base + chunk, chunk)], iv1)
        for c in range(nchunks):
            cur, nxt = c % 2, (c + 1) % 2
            copies[cur].wait()
            if c + 1 < nchunks:
                copies[nxt] = pltpu.async_copy(g_hbm.at[iv[nxt]], rv[nxt],
                                               sg[nxt])
            pltpu.sync_copy(rv[cur], out_hbm.at[pl.ds(base + c * chunk,
                                                      chunk)])
            if c + 2 < nchunks:
                pltpu.sync_copy(idx_hbm.at[pl.ds(base + (c + 2) * chunk,
                                                 chunk)], iv[cur])

    return k(g, idx)


def kernel(xyz, points, W1, b1, W2, b2, W3, b3):
    B, N, _ = xyz.shape
    f32 = jnp.float32
    init_id = jax.random.randint(jax.random.key(42), (B,), 0, N - 1)
    init_f = init_id.astype(f32).reshape(B, 1)
    xyz_t = jnp.transpose(xyz, (0, 2, 1))

    cshape = jax.ShapeDtypeStruct((B, _NP), f32)
    cx, cy, cz = pl.pallas_call(
        _fps_body,
        out_shape=[cshape, cshape, cshape],
    )(xyz_t, init_f)

    gidx = pl.pallas_call(
        _ballq_body,
        grid=(B, _NP // _CB),
        in_specs=[
            pl.BlockSpec((1, 3, N), lambda b, j: (b, 0, 0)),
            pl.BlockSpec((1, 1, _NP), lambda b, j: (b, 0, 0)),
            pl.BlockSpec((1, 1, _NP), lambda b, j: (b, 0, 0)),
            pl.BlockSpec((1, 1, _NP), lambda b, j: (b, 0, 0)),
        ],
        out_specs=pl.BlockSpec((1, _NS, _NP), lambda b, j: (b, 0, 0)),
        out_shape=jax.ShapeDtypeStruct((B, _NS, _NP), jnp.int32),
        scratch_shapes=[pltpu.VMEM((_CB, _N), f32)],
    )(xyz_t, cx.reshape(B, 1, _NP), cy.reshape(B, 1, _NP),
      cz.reshape(B, 1, _NP))
    gidx_flat = gidx.transpose(0, 2, 1).reshape(-1)

    W1p = jnp.pad(W1, ((0, 0), (0, 64)))
    b1p = jnp.pad(b1, (0, 64))
    W2p = jnp.pad(W2, ((0, 64), (0, 0)))

    T = jnp.concatenate([xyz, points], axis=-1).reshape(B * N, 3 + _DP)
    RB = 2048
    G = pl.pallas_call(
        _mm_body,
        grid=(B * N // RB,),
        in_specs=[
            pl.BlockSpec((RB, 3 + _DP), lambda i: (i, 0)),
            pl.BlockSpec((3 + _DP, 128), lambda i: (0, 0)),
        ],
        out_specs=pl.BlockSpec((RB, 128), lambda i: (i, 0)),
        out_shape=jax.ShapeDtypeStruct((B * N, 128), f32),
    )(T, W1p)

    X1 = _sc_gather(G, gidx_flat)

    MB = _NS * _NS
    out = pl.pallas_call(
        _mlp_body,
        grid=(B * _NP * _NS // MB,),
        in_specs=[
            pl.BlockSpec((MB, 128), lambda i: (i, 0)),
            pl.BlockSpec((1, 128), lambda i: (0, 0)),
            pl.BlockSpec((128, 64), lambda i: (0, 0)),
            pl.BlockSpec((1, 64), lambda i: (0, 0)),
            pl.BlockSpec((64, 128), lambda i: (0, 0)),
            pl.BlockSpec((1, 128), lambda i: (0, 0)),
        ],
        out_specs=pl.BlockSpec((_NS, 128), lambda i: (i, 0)),
        out_shape=jax.ShapeDtypeStruct((B * _NP, 128), f32),
    )(X1, b1p.reshape(1, 128), W2p, b2.reshape(1, 64), W3, b3.reshape(1, 128))

    cent_xyz = jnp.stack([cx, cy, cz], axis=-1)
    return (cent_xyz, out.reshape(B, _NP, 128))


# speedup vs baseline: 15.0740x; 1.0248x over previous
import functools

import numpy as np
import jax
import jax.numpy as jnp
from jax import lax
from jax.experimental import pallas as pl
from jax.experimental.pallas import tpu as pltpu
from jax.experimental.pallas import tpu_sc as plsc

_B, _N, _DP = 8, 4096, 64
_NP = 512
_NS = 32
_CB = 128
_T04 = np.float32(0.2 ** 2)


def _fps_body(xyz_ref, init_ref, cx_ref, cy_ref, cz_ref, dist_ref, dbuf0,
              dbuf1, dsem):
    X = xyz_ref[:, 0, :]
    Y = xyz_ref[:, 1, :]
    Z = xyz_ref[:, 2, :]
    iota = lax.broadcasted_iota(jnp.int32, (_B, _N), 1)
    slot = lax.broadcasted_iota(jnp.int32, (_B, _NP), 1)
    zero = jnp.zeros((_B, _N), jnp.float32)
    zc = jnp.zeros((_B, _NP), jnp.float32)

    def pick(sel):
        px = jnp.sum(jnp.where(sel, X, zero), axis=1, keepdims=True)
        py = jnp.sum(jnp.where(sel, Y, zero), axis=1, keepdims=True)
        pz = jnp.sum(jnp.where(sel, Z, zero), axis=1, keepdims=True)
        return px, py, pz

    init_i = init_ref[...].astype(jnp.int32)
    px, py, pz = pick(iota == init_i)
    cxs = jnp.where(slot == 0, px, zc)
    cys = jnp.where(slot == 0, py, zc)
    czs = jnp.where(slot == 0, pz, zc)
    mask = jnp.ones((_B, _N), jnp.float32)

    def dist_row(px, py, pz):
        dx = X - px
        dy = Y - py
        dz = Z - pz
        return jnp.sqrt(dx * dx + dy * dy + dz * dz)

    bufs = (dbuf0, dbuf1)

    def drain(s, i):
        pltpu.make_async_copy(bufs[s], dist_ref.at[:, pl.ds(i, 1), :],
                              dsem.at[s]).wait()

    def emit(s, i, d):
        bufs[s][...] = jnp.minimum(d, _T04)[:, None, :]
        pltpu.make_async_copy(bufs[s], dist_ref.at[:, pl.ds(i, 1), :],
                              dsem.at[s]).start()

    def body(i, carry):
        px, py, pz, cxs, cys, czs, mask = carry
        d = dist_row(px, py, pz)
        par = lax.rem(i, 2)
        for s in (0, 1):
            @pl.when((par == s) & (i >= 2))
            def _(s=s):
                drain(s, i - 2)

            @pl.when(par == s)
            def _(s=s):
                emit(s, i, d)

        dm = d * mask
        mx = jnp.max(dm, axis=1, keepdims=True)
        idx = jnp.min(jnp.where(dm == mx, iota, jnp.int32(_N)), axis=1,
                      keepdims=True)
        npx, npy, npz = pick(iota == idx)
        nmask = jnp.minimum(dm * mask * jnp.float32(1e11), mask)
        w = slot == (i + 1)
        cxs = jnp.where(w, npx, cxs)
        cys = jnp.where(w, npy, cys)
        czs = jnp.where(w, npz, czs)
        return (npx, npy, npz, cxs, cys, czs, nmask)

    carry = (px, py, pz, cxs, cys, czs, mask)
    px, py, pz, cxs, cys, czs, _ = lax.fori_loop(0, _NP - 1, body, carry)
    drain(0, _NP - 2)
    drain(1, _NP - 3)
    emit(0, _NP - 1, dist_row(px, py, pz))
    drain(0, _NP - 1)
    cx_ref[...] = cxs
    cy_ref[...] = cys
    cz_ref[...] = czs


def _ballq_body(dist_ref, out_ref, key_scr):
    b = pl.program_id(0)
    j = pl.program_id(1)
    c0 = pl.multiple_of(j * _CB, 128)
    dc = dist_ref[0]
    iota = lax.broadcasted_iota(jnp.int32, (_CB, _N), 1)
    key = jnp.where(dc < _T04, dc, jnp.float32(1.0) + iota.astype(jnp.float32))
    key_scr[...] = key
    base = b * _N
    krow = lax.broadcasted_iota(jnp.int32, (_NS, _CB), 0)
    one = jnp.float32(1.0)
    inf = jnp.float32(jnp.inf)

    def p1_cond(carry):
        _, _, mn = carry
        return jnp.min(mn) < one

    def p1_body(carry):
        acc, cnt, mn = carry
        key = key_scr[...]
        idx = jnp.min(jnp.where(key == mn[:, None], iota, jnp.int32(_N)),
                      axis=1)
        act = mn < one
        acc = jnp.where((krow == cnt[None, :]) & act[None, :],
                        (idx + base)[None, :], acc)
        cnt = cnt + act.astype(jnp.int32)
        key = jnp.where((iota == idx[:, None]) & act[:, None], inf, key)
        key_scr[...] = key
        return (acc, cnt, jnp.min(key, axis=1))

    acc0 = jnp.zeros((_NS, _CB), jnp.int32)
    cnt0 = jnp.zeros((_CB,), jnp.int32)
    mn0 = jnp.min(key, axis=1)
    acc, cnt, _ = lax.while_loop(p1_cond, p1_body, (acc0, cnt0, mn0))

    iota64 = lax.broadcasted_iota(jnp.int32, (_CB, 64), 1)

    def p2_body(k, carry):
        acc, cnt = carry
        sub = key_scr[:, :64]
        mn = jnp.min(sub, axis=1)
        idx = jnp.min(jnp.where(sub == mn[:, None], iota64, jnp.int32(_N)),
                      axis=1)
        acc = jnp.where(krow == cnt[None, :], (idx + base)[None, :], acc)
        cnt = cnt + 1
        key_scr[:, :64] = jnp.where(iota64 == idx[:, None], inf, sub)
        return (acc, cnt)

    acc, _ = lax.fori_loop(0, _NS, p2_body, (acc, cnt))
    out_ref[0, :, pl.ds(c0, _CB)] = acc


def _mm_body(t_ref, w_ref, out_ref):
    out_ref[...] = lax.dot_general(
        t_ref[...], w_ref[...], (((1,), (0,)), ((), ())),
        preferred_element_type=jnp.float32)


def _mlp_body(x_ref, b1_ref, w2_ref, b2_ref, w3_ref, b3_ref, out_ref):
    dn = (((1,), (0,)), ((), ()))
    h = jnp.maximum(x_ref[...] + b1_ref[...], jnp.float32(0.0))
    h = lax.dot_general(h, w2_ref[...], dn,
                        preferred_element_type=jnp.float32)
    h = jnp.maximum(h + b2_ref[...], jnp.float32(0.0))
    h = lax.dot_general(h, w3_ref[...], dn,
                        preferred_element_type=jnp.float32)
    h = jnp.maximum(h + b3_ref[...], jnp.float32(0.0))
    out_ref[...] = jnp.max(h.reshape(_NS, _NS, 128), axis=1)


def _sc_gather(g, idx):
    info = plsc.get_sparse_core_info()
    ncores = info.num_cores
    nw = ncores * info.num_subcores
    rows = idx.shape[0]
    per_w = rows // nw
    chunk = 128
    nchunks = per_w // chunk
    mesh = plsc.VectorSubcoreMesh(core_axis_name="c", subcore_axis_name="s")

    @functools.partial(
        pl.kernel, mesh=mesh,
        out_type=jax.ShapeDtypeStruct((rows, g.shape[1]), jnp.float32),
        scratch_types=[
            pltpu.VMEM((chunk,), jnp.int32),
            pltpu.VMEM((chunk,), jnp.int32),
            pltpu.VMEM((chunk, g.shape[1]), jnp.float32),
            pltpu.VMEM((chunk, g.shape[1]), jnp.float32),
            pltpu.SemaphoreType.DMA,
            pltpu.SemaphoreType.DMA,
        ],
    )
    def k(g_hbm, idx_hbm, out_hbm, iv0, iv1, rv0, rv1, sg0, sg1):
        wid = lax.axis_index("s") * ncores + lax.axis_index("c")
        base = wid * per_w
        iv = (iv0, iv1)
        rv = (rv0, rv1)
        sg = (sg0, sg1)

        pltpu.sync_copy(idx_hbm.at[pl.ds(base, chunk)], iv0)
        copies = [pltpu.async_copy(g_hbm.at[iv0], rv0, sg0), None]
        pltpu.sync_copy(idx_hbm.at[pl.ds(base + chunk, chunk)], iv1)
        for c in range(nchunks):
            cur, nxt = c % 2, (c + 1) % 2
            copies[cur].wait()
            if c + 1 < nchunks:
                copies[nxt] = pltpu.async_copy(g_hbm.at[iv[nxt]], rv[nxt],
                                               sg[nxt])
            pltpu.sync_copy(rv[cur], out_hbm.at[pl.ds(base + c * chunk,
                                                      chunk)])
            if c + 2 < nchunks:
                pltpu.sync_copy(idx_hbm.at[pl.ds(base + (c + 2) * chunk,
                                                 chunk)], iv[cur])

    return k(g, idx)


def kernel(xyz, points, W1, b1, W2, b2, W3, b3):
    B, N, _ = xyz.shape
    f32 = jnp.float32
    init_id = jax.random.randint(jax.random.key(42), (B,), 0, N - 1)
    init_f = init_id.astype(f32).reshape(B, 1)
    xyz_t = jnp.transpose(xyz, (0, 2, 1))

    cshape = jax.ShapeDtypeStruct((B, _NP), f32)
    cx, cy, cz, dist = pl.pallas_call(
        _fps_body,
        out_shape=[cshape, cshape, cshape,
                   jax.ShapeDtypeStruct((B, _NP, N), f32)],
        out_specs=[pl.BlockSpec(), pl.BlockSpec(), pl.BlockSpec(),
                   pl.BlockSpec(memory_space=pltpu.HBM)],
        scratch_shapes=[pltpu.VMEM((B, 1, N), f32),
                        pltpu.VMEM((B, 1, N), f32),
                        pltpu.SemaphoreType.DMA((2,))],
    )(xyz_t, init_f)

    gidx = pl.pallas_call(
        _ballq_body,
        grid=(B, _NP // _CB),
        in_specs=[pl.BlockSpec((1, _CB, N), lambda b, j: (b, j, 0))],
        out_specs=pl.BlockSpec((1, _NS, _NP), lambda b, j: (b, 0, 0)),
        out_shape=jax.ShapeDtypeStruct((B, _NS, _NP), jnp.int32),
        scratch_shapes=[pltpu.VMEM((_CB, _N), f32)],
    )(dist)
    gidx_flat = gidx.transpose(0, 2, 1).reshape(-1)

    W1p = jnp.pad(W1, ((0, 0), (0, 64)))
    b1p = jnp.pad(b1, (0, 64))
    W2p = jnp.pad(W2, ((0, 64), (0, 0)))

    T = jnp.concatenate([xyz, points], axis=-1).reshape(B * N, 3 + _DP)
    RB = 2048
    G = pl.pallas_call(
        _mm_body,
        grid=(B * N // RB,),
        in_specs=[
            pl.BlockSpec((RB, 3 + _DP), lambda i: (i, 0)),
            pl.BlockSpec((3 + _DP, 128), lambda i: (0, 0)),
        ],
        out_specs=pl.BlockSpec((RB, 128), lambda i: (i, 0)),
        out_shape=jax.ShapeDtypeStruct((B * N, 128), f32),
    )(T, W1p)

    X1 = _sc_gather(G, gidx_flat)

    MB = _NS * _NS
    out = pl.pallas_call(
        _mlp_body,
        grid=(B * _NP * _NS // MB,),
        in_specs=[
            pl.BlockSpec((MB, 128), lambda i: (i, 0)),
            pl.BlockSpec((1, 128), lambda i: (0, 0)),
            pl.BlockSpec((128, 64), lambda i: (0, 0)),
            pl.BlockSpec((1, 64), lambda i: (0, 0)),
            pl.BlockSpec((64, 128), lambda i: (0, 0)),
            pl.BlockSpec((1, 128), lambda i: (0, 0)),
        ],
        out_specs=pl.BlockSpec((_NS, 128), lambda i: (i, 0)),
        out_shape=jax.ShapeDtypeStruct((B * _NP, 128), f32),
    )(X1, b1p.reshape(1, 128), W2p, b2.reshape(1, 64), W3, b3.reshape(1, 128))

    cent_xyz = jnp.stack([cx, cy, cz], axis=-1)
    return (cent_xyz, out.reshape(B, _NP, 128))


# speedup vs baseline: 16.0381x; 1.0640x over previous
import functools

import numpy as np
import jax
import jax.numpy as jnp
from jax import lax
from jax.experimental import pallas as pl
from jax.experimental.pallas import tpu as pltpu
from jax.experimental.pallas import tpu_sc as plsc

_B, _N, _DP = 8, 4096, 64
_NP = 512
_NS = 32
_CB = 128
_T04 = np.float32(0.2 ** 2)


def _fps_body(xyz_ref, init_ref, cx_ref, cy_ref, cz_ref, dist_ref, dbuf0,
              dbuf1, dsem):
    X = xyz_ref[:, 0, :]
    Y = xyz_ref[:, 1, :]
    Z = xyz_ref[:, 2, :]
    iota = lax.broadcasted_iota(jnp.int32, (_B, _N), 1)
    slot = lax.broadcasted_iota(jnp.int32, (_B, _NP), 1)
    zero = jnp.zeros((_B, _N), jnp.float32)
    zc = jnp.zeros((_B, _NP), jnp.float32)

    def pick(sel):
        px = jnp.sum(jnp.where(sel, X, zero), axis=1, keepdims=True)
        py = jnp.sum(jnp.where(sel, Y, zero), axis=1, keepdims=True)
        pz = jnp.sum(jnp.where(sel, Z, zero), axis=1, keepdims=True)
        return px, py, pz

    init_i = init_ref[...].astype(jnp.int32)
    px, py, pz = pick(iota == init_i)
    cxs = jnp.where(slot == 0, px, zc)
    cys = jnp.where(slot == 0, py, zc)
    czs = jnp.where(slot == 0, pz, zc)
    mask = jnp.ones((_B, _N), jnp.float32)

    def dist_row(px, py, pz):
        dx = X - px
        dy = Y - py
        dz = Z - pz
        return jnp.sqrt(dx * dx + dy * dy + dz * dz)

    bufs = (dbuf0, dbuf1)

    def drain(s, i):
        pltpu.make_async_copy(bufs[s], dist_ref.at[:, pl.ds(i, 1), :],
                              dsem.at[s]).wait()

    def emit(s, i, d):
        bufs[s][...] = jnp.minimum(d, _T04)[:, None, :]
        pltpu.make_async_copy(bufs[s], dist_ref.at[:, pl.ds(i, 1), :],
                              dsem.at[s]).start()

    def body(i, carry):
        px, py, pz, cxs, cys, czs, mask = carry
        d = dist_row(px, py, pz)
        par = lax.rem(i, 2)
        for s in (0, 1):
            @pl.when((par == s) & (i >= 2))
            def _(s=s):
                drain(s, i - 2)

            @pl.when(par == s)
            def _(s=s):
                emit(s, i, d)

        dm = d * mask
        mx = jnp.max(dm, axis=1, keepdims=True)
        idx = jnp.min(jnp.where(dm == mx, iota, jnp.int32(_N)), axis=1,
                      keepdims=True)
        npx, npy, npz = pick(iota == idx)
        nmask = jnp.minimum(dm * mask * jnp.float32(1e11), mask)
        w = slot == (i + 1)
        cxs = jnp.where(w, npx, cxs)
        cys = jnp.where(w, npy, cys)
        czs = jnp.where(w, npz, czs)
        return (npx, npy, npz, cxs, cys, czs, nmask)

    carry = (px, py, pz, cxs, cys, czs, mask)
    px, py, pz, cxs, cys, czs, _ = lax.fori_loop(0, _NP - 1, body, carry)
    drain(0, _NP - 2)
    drain(1, _NP - 3)
    emit(0, _NP - 1, dist_row(px, py, pz))
    drain(0, _NP - 1)
    cx_ref[...] = cxs
    cy_ref[...] = cys
    cz_ref[...] = czs


def _ballq_body(dist_ref, out_ref, key_scr):
    b = pl.program_id(0)
    j = pl.program_id(1)
    c0 = pl.multiple_of(j * _CB, 128)
    dc = dist_ref[0]
    iota = lax.broadcasted_iota(jnp.int32, (_CB, _N), 1)
    key = jnp.where(dc < _T04, dc, jnp.float32(1.0) + iota.astype(jnp.float32))
    key_scr[...] = key
    base = b * _N
    krow = lax.broadcasted_iota(jnp.int32, (_NS, _CB), 0)
    one = jnp.float32(1.0)
    inf = jnp.float32(jnp.inf)

    def p1_cond(carry):
        _, _, mn = carry
        return jnp.min(mn) < one

    def p1_body(carry):
        acc, cnt, mn = carry
        key = key_scr[...]
        idx = jnp.min(jnp.where(key == mn[:, None], iota, jnp.int32(_N)),
                      axis=1)
        act = mn < one
        acc = jnp.where((krow == cnt[None, :]) & act[None, :],
                        (idx + base)[None, :], acc)
        cnt = cnt + act.astype(jnp.int32)
        key = jnp.where((iota == idx[:, None]) & act[:, None], inf, key)
        key_scr[...] = key
        return (acc, cnt, jnp.min(key, axis=1))

    acc0 = jnp.zeros((_NS, _CB), jnp.int32)
    cnt0 = jnp.zeros((_CB,), jnp.int32)
    mn0 = jnp.min(key, axis=1)
    acc, cnt, _ = lax.while_loop(p1_cond, p1_body, (acc0, cnt0, mn0))

    iota64 = lax.broadcasted_iota(jnp.int32, (_CB, 64), 1)

    def p2_body(k, carry):
        acc, cnt = carry
        sub = key_scr[:, :64]
        mn = jnp.min(sub, axis=1)
        idx = jnp.min(jnp.where(sub == mn[:, None], iota64, jnp.int32(_N)),
                      axis=1)
        acc = jnp.where(krow == cnt[None, :], (idx + base)[None, :], acc)
        cnt = cnt + 1
        key_scr[:, :64] = jnp.where(iota64 == idx[:, None], inf, sub)
        return (acc, cnt)

    acc, _ = lax.fori_loop(0, _NS, p2_body, (acc, cnt))
    out_ref[0, :, pl.ds(c0, _CB)] = acc


def _mm_body(t_ref, w_ref, out_ref):
    out_ref[...] = lax.dot_general(
        t_ref[...], w_ref[...], (((1,), (0,)), ((), ())),
        preferred_element_type=jnp.float32)


def _mlp_body(x_ref, b1_ref, w2_ref, b2_ref, w3_ref, b3_ref, out_ref):
    dn = (((1,), (0,)), ((), ()))
    h = jnp.maximum(x_ref[...] + b1_ref[...], jnp.float32(0.0))
    h = lax.dot_general(h, w2_ref[...], dn,
                        preferred_element_type=jnp.float32)
    h = jnp.maximum(h + b2_ref[...], jnp.float32(0.0))
    h = lax.dot_general(h, w3_ref[...], dn,
                        preferred_element_type=jnp.float32)
    h = jnp.maximum(h + b3_ref[...], jnp.float32(0.0))
    out_ref[...] = jnp.max(h.reshape(-1, _NS, 128), axis=1)


def _sc_gather(g, idx):
    info = plsc.get_sparse_core_info()
    ncores = info.num_cores
    nw = ncores * info.num_subcores
    rows = idx.shape[0]
    per_w = rows // nw
    chunk = 128
    nchunks = per_w // chunk
    mesh = plsc.VectorSubcoreMesh(core_axis_name="c", subcore_axis_name="s")

    @functools.partial(
        pl.kernel, mesh=mesh,
        out_type=jax.ShapeDtypeStruct((rows, g.shape[1]), jnp.float32),
        scratch_types=[
            pltpu.VMEM((chunk,), jnp.int32),
            pltpu.VMEM((chunk,), jnp.int32),
            pltpu.VMEM((chunk, g.shape[1]), jnp.float32),
            pltpu.VMEM((chunk, g.shape[1]), jnp.float32),
            pltpu.SemaphoreType.DMA,
            pltpu.SemaphoreType.DMA,
        ],
    )
    def k(g_hbm, idx_hbm, out_hbm, iv0, iv1, rv0, rv1, sg0, sg1):
        wid = lax.axis_index("s") * ncores + lax.axis_index("c")
        base = wid * per_w
        iv = (iv0, iv1)
        rv = (rv0, rv1)
        sg = (sg0, sg1)

        pltpu.sync_copy(idx_hbm.at[pl.ds(base, chunk)], iv0)
        copies = [pltpu.async_copy(g_hbm.at[iv0], rv0, sg0), None]
        pltpu.sync_copy(idx_hbm.at[pl.ds(base + chunk, chunk)], iv1)
        for c in range(nchunks):
            cur, nxt = c % 2, (c + 1) % 2
            copies[cur].wait()
            if c + 1 < nchunks:
                copies[nxt] = pltpu.async_copy(g_hbm.at[iv[nxt]], rv[nxt],
                                               sg[nxt])
            pltpu.sync_copy(rv[cur], out_hbm.at[pl.ds(base + c * chunk,
                                                      chunk)])
            if c + 2 < nchunks:
                pltpu.sync_copy(idx_hbm.at[pl.ds(base + (c + 2) * chunk,
                                                 chunk)], iv[cur])

    return k(g, idx)


def kernel(xyz, points, W1, b1, W2, b2, W3, b3):
    B, N, _ = xyz.shape
    f32 = jnp.float32
    init_id = jax.random.randint(jax.random.key(42), (B,), 0, N - 1)
    init_f = init_id.astype(f32).reshape(B, 1)
    xyz_t = jnp.transpose(xyz, (0, 2, 1))

    cshape = jax.ShapeDtypeStruct((B, _NP), f32)
    cx, cy, cz, dist = pl.pallas_call(
        _fps_body,
        out_shape=[cshape, cshape, cshape,
                   jax.ShapeDtypeStruct((B, _NP, N), f32)],
        out_specs=[pl.BlockSpec(), pl.BlockSpec(), pl.BlockSpec(),
                   pl.BlockSpec(memory_space=pltpu.HBM)],
        scratch_shapes=[pltpu.VMEM((B, 1, N), f32),
                        pltpu.VMEM((B, 1, N), f32),
                        pltpu.SemaphoreType.DMA((2,))],
    )(xyz_t, init_f)

    gidx = pl.pallas_call(
        _ballq_body,
        grid=(B, _NP // _CB),
        in_specs=[pl.BlockSpec((1, _CB, N), lambda b, j: (b, j, 0))],
        out_specs=pl.BlockSpec((1, _NS, _NP), lambda b, j: (b, 0, 0)),
        out_shape=jax.ShapeDtypeStruct((B, _NS, _NP), jnp.int32),
        scratch_shapes=[pltpu.VMEM((_CB, _N), f32)],
    )(dist)
    gidx_flat = gidx.transpose(0, 2, 1).reshape(-1)

    W1p = jnp.pad(W1, ((0, 0), (0, 64)))
    b1p = jnp.pad(b1, (0, 64))
    W2p = jnp.pad(W2, ((0, 64), (0, 0)))

    T = jnp.concatenate([xyz, points], axis=-1).reshape(B * N, 3 + _DP)
    RB = 4096
    G = pl.pallas_call(
        _mm_body,
        grid=(B * N // RB,),
        in_specs=[
            pl.BlockSpec((RB, 3 + _DP), lambda i: (i, 0)),
            pl.BlockSpec((3 + _DP, 128), lambda i: (0, 0)),
        ],
        out_specs=pl.BlockSpec((RB, 128), lambda i: (i, 0)),
        out_shape=jax.ShapeDtypeStruct((B * N, 128), f32),
    )(T, W1p)

    X1 = _sc_gather(G, gidx_flat)

    MB = 4096
    out = pl.pallas_call(
        _mlp_body,
        grid=(B * _NP * _NS // MB,),
        in_specs=[
            pl.BlockSpec((MB, 128), lambda i: (i, 0)),
            pl.BlockSpec((1, 128), lambda i: (0, 0)),
            pl.BlockSpec((128, 64), lambda i: (0, 0)),
            pl.BlockSpec((1, 64), lambda i: (0, 0)),
            pl.BlockSpec((64, 128), lambda i: (0, 0)),
            pl.BlockSpec((1, 128), lambda i: (0, 0)),
        ],
        out_specs=pl.BlockSpec((128, 128), lambda i: (i, 0)),
        out_shape=jax.ShapeDtypeStruct((B * _NP, 128), f32),
    )(X1, b1p.reshape(1, 128), W2p, b2.reshape(1, 64), W3, b3.reshape(1, 128))

    cent_xyz = jnp.stack([cx, cy, cz], axis=-1)
    return (cent_xyz, out.reshape(B, _NP, 128))


# speedup vs baseline: 16.7351x; 1.0435x over previous
import functools

import numpy as np
import jax
import jax.numpy as jnp
from jax import lax
from jax.experimental import pallas as pl
from jax.experimental.pallas import tpu as pltpu
from jax.experimental.pallas import tpu_sc as plsc

_B, _N, _DP = 8, 4096, 64
_NP = 512
_NS = 32
_CB = 128
_T04 = np.float32(0.2 ** 2)


def _fps_body(xyz_ref, init_ref, t_ref, w_ref, cx_ref, cy_ref, cz_ref,
              dist_ref, g_ref, dbuf0, dbuf1, dsem):
    g_ref[...] = lax.dot_general(
        t_ref[...], w_ref[...], (((1,), (0,)), ((), ())),
        preferred_element_type=jnp.float32)
    X = xyz_ref[:, 0, :]
    Y = xyz_ref[:, 1, :]
    Z = xyz_ref[:, 2, :]
    iota = lax.broadcasted_iota(jnp.int32, (_B, _N), 1)
    slot = lax.broadcasted_iota(jnp.int32, (_B, _NP), 1)
    zero = jnp.zeros((_B, _N), jnp.float32)
    zc = jnp.zeros((_B, _NP), jnp.float32)

    def pick(sel):
        px = jnp.sum(jnp.where(sel, X, zero), axis=1, keepdims=True)
        py = jnp.sum(jnp.where(sel, Y, zero), axis=1, keepdims=True)
        pz = jnp.sum(jnp.where(sel, Z, zero), axis=1, keepdims=True)
        return px, py, pz

    init_i = init_ref[...].astype(jnp.int32)
    px, py, pz = pick(iota == init_i)
    cxs = jnp.where(slot == 0, px, zc)
    cys = jnp.where(slot == 0, py, zc)
    czs = jnp.where(slot == 0, pz, zc)
    mask = jnp.ones((_B, _N), jnp.float32)

    def dist_row(px, py, pz):
        dx = X - px
        dy = Y - py
        dz = Z - pz
        return jnp.sqrt(dx * dx + dy * dy + dz * dz)

    bufs = (dbuf0, dbuf1)

    def drain(s, i):
        pltpu.make_async_copy(bufs[s], dist_ref.at[:, pl.ds(i, 1), :],
                              dsem.at[s]).wait()

    def emit(s, i, d):
        bufs[s][...] = jnp.minimum(d, _T04)[:, None, :]
        pltpu.make_async_copy(bufs[s], dist_ref.at[:, pl.ds(i, 1), :],
                              dsem.at[s]).start()

    def body(i, carry):
        px, py, pz, cxs, cys, czs, mask = carry
        d = dist_row(px, py, pz)
        par = lax.rem(i, 2)
        for s in (0, 1):
            @pl.when((par == s) & (i >= 2))
            def _(s=s):
                drain(s, i - 2)

            @pl.when(par == s)
            def _(s=s):
                emit(s, i, d)

        dm = d * mask
        mx = jnp.max(dm, axis=1, keepdims=True)
        idx = jnp.min(jnp.where(dm == mx, iota, jnp.int32(_N)), axis=1,
                      keepdims=True)
        npx, npy, npz = pick(iota == idx)
        nmask = jnp.minimum(dm * mask * jnp.float32(1e11), mask)
        w = slot == (i + 1)
        cxs = jnp.where(w, npx, cxs)
        cys = jnp.where(w, npy, cys)
        czs = jnp.where(w, npz, czs)
        return (npx, npy, npz, cxs, cys, czs, nmask)

    carry = (px, py, pz, cxs, cys, czs, mask)
    px, py, pz, cxs, cys, czs, _ = lax.fori_loop(0, _NP - 1, body, carry)
    drain(0, _NP - 2)
    drain(1, _NP - 3)
    emit(0, _NP - 1, dist_row(px, py, pz))
    drain(0, _NP - 1)
    cx_ref[...] = cxs
    cy_ref[...] = cys
    cz_ref[...] = czs


def _ballq_body(dist_ref, out_ref, key_scr):
    b = pl.program_id(0)
    j = pl.program_id(1)
    c0 = pl.multiple_of(j * _CB, 128)
    dc = dist_ref[0]
    iota = lax.broadcasted_iota(jnp.int32, (_CB, _N), 1)
    key = jnp.where(dc < _T04, dc, jnp.float32(1.0) + iota.astype(jnp.float32))
    key_scr[...] = key
    base = b * _N
    kcol = lax.broadcasted_iota(jnp.int32, (_CB, _NS), 1)
    one = jnp.float32(1.0)
    inf = jnp.float32(jnp.inf)

    def p1_cond(carry):
        _, _, mn = carry
        return jnp.min(mn) < one

    def p1_body(carry):
        acc, cnt, mn = carry
        key = key_scr[...]
        idx = jnp.min(jnp.where(key == mn[:, None], iota, jnp.int32(_N)),
                      axis=1)
        act = mn < one
        acc = jnp.where((kcol == cnt[:, None]) & act[:, None],
                        (idx + base)[:, None], acc)
        cnt = cnt + act.astype(jnp.int32)
        key = jnp.where((iota == idx[:, None]) & act[:, None], inf, key)
        key_scr[...] = key
        return (acc, cnt, jnp.min(key, axis=1))

    acc0 = jnp.zeros((_CB, _NS), jnp.int32)
    cnt0 = jnp.zeros((_CB,), jnp.int32)
    mn0 = jnp.min(key, axis=1)
    acc, cnt, _ = lax.while_loop(p1_cond, p1_body, (acc0, cnt0, mn0))

    iota64 = lax.broadcasted_iota(jnp.int32, (_CB, 64), 1)

    def p2_body(k, carry):
        acc, cnt = carry
        sub = key_scr[:, :64]
        mn = jnp.min(sub, axis=1)
        idx = jnp.min(jnp.where(sub == mn[:, None], iota64, jnp.int32(_N)),
                      axis=1)
        acc = jnp.where(kcol == cnt[:, None], (idx + base)[:, None], acc)
        cnt = cnt + 1
        key_scr[:, :64] = jnp.where(iota64 == idx[:, None], inf, sub)
        return (acc, cnt)

    acc, _ = lax.fori_loop(0, _NS, p2_body, (acc, cnt))
    out_ref[0, :, :] = acc


def _mm_body(t_ref, w_ref, out_ref):
    out_ref[...] = lax.dot_general(
        t_ref[...], w_ref[...], (((1,), (0,)), ((), ())),
        preferred_element_type=jnp.float32)


def _mlp_body(x_ref, b1_ref, w2_ref, b2_ref, w3_ref, b3_ref, out_ref):
    dn = (((1,), (0,)), ((), ()))
    h = jnp.maximum(x_ref[...] + b1_ref[...], jnp.float32(0.0))
    h = lax.dot_general(h, w2_ref[...], dn,
                        preferred_element_type=jnp.float32)
    h = jnp.maximum(h + b2_ref[...], jnp.float32(0.0))
    h = lax.dot_general(h, w3_ref[...], dn,
                        preferred_element_type=jnp.float32)
    h = jnp.maximum(h + b3_ref[...], jnp.float32(0.0))
    out_ref[...] = jnp.max(h.reshape(-1, _NS, 128), axis=1)


def _sc_gather(g, idx):
    info = plsc.get_sparse_core_info()
    ncores = info.num_cores
    nw = ncores * info.num_subcores
    rows = idx.shape[0]
    per_w = rows // nw
    chunk = 128
    nchunks = per_w // chunk
    mesh = plsc.VectorSubcoreMesh(core_axis_name="c", subcore_axis_name="s")

    @functools.partial(
        pl.kernel, mesh=mesh,
        out_type=jax.ShapeDtypeStruct((rows, g.shape[1]), jnp.float32),
        scratch_types=[
            pltpu.VMEM((chunk,), jnp.int32),
            pltpu.VMEM((chunk,), jnp.int32),
            pltpu.VMEM((chunk, g.shape[1]), jnp.float32),
            pltpu.VMEM((chunk, g.shape[1]), jnp.float32),
            pltpu.SemaphoreType.DMA,
            pltpu.SemaphoreType.DMA,
        ],
    )
    def k(g_hbm, idx_hbm, out_hbm, iv0, iv1, rv0, rv1, sg0, sg1):
        wid = lax.axis_index("s") * ncores + lax.axis_index("c")
        base = wid * per_w
        iv = (iv0, iv1)
        rv = (rv0, rv1)
        sg = (sg0, sg1)

        pltpu.sync_copy(idx_hbm.at[pl.ds(base, chunk)], iv0)
        copies = [pltpu.async_copy(g_hbm.at[iv0], rv0, sg0), None]
        pltpu.sync_copy(idx_hbm.at[pl.ds(base + chunk, chunk)], iv1)
        for c in range(nchunks):
            cur, nxt = c % 2, (c + 1) % 2
            copies[cur].wait()
            if c + 1 < nchunks:
                copies[nxt] = pltpu.async_copy(g_hbm.at[iv[nxt]], rv[nxt],
                                               sg[nxt])
            pltpu.sync_copy(rv[cur], out_hbm.at[pl.ds(base + c * chunk,
                                                      chunk)])
            if c + 2 < nchunks:
                pltpu.sync_copy(idx_hbm.at[pl.ds(base + (c + 2) * chunk,
                                                 chunk)], iv[cur])

    return k(g, idx)


def kernel(xyz, points, W1, b1, W2, b2, W3, b3):
    B, N, _ = xyz.shape
    f32 = jnp.float32
    init_id = jax.random.randint(jax.random.key(42), (B,), 0, N - 1)
    init_f = init_id.astype(f32).reshape(B, 1)
    xyz_t = jnp.transpose(xyz, (0, 2, 1))

    W1p = jnp.pad(W1, ((0, 0), (0, 64)))
    T = jnp.concatenate([xyz, points], axis=-1).reshape(B * N, 3 + _DP)
    cshape = jax.ShapeDtypeStruct((B, _NP), f32)
    cx, cy, cz, dist, G = pl.pallas_call(
        _fps_body,
        out_shape=[cshape, cshape, cshape,
                   jax.ShapeDtypeStruct((B, _NP, N), f32),
                   jax.ShapeDtypeStruct((B * N, 128), f32)],
        out_specs=[pl.BlockSpec(), pl.BlockSpec(), pl.BlockSpec(),
                   pl.BlockSpec(memory_space=pltpu.HBM), pl.BlockSpec()],
        scratch_shapes=[pltpu.VMEM((B, 1, N), f32),
                        pltpu.VMEM((B, 1, N), f32),
                        pltpu.SemaphoreType.DMA((2,))],
    )(xyz_t, init_f, T, W1p)

    gidx = pl.pallas_call(
        _ballq_body,
        grid=(B, _NP // _CB),
        in_specs=[pl.BlockSpec((1, _CB, N), lambda b, j: (b, j, 0))],
        out_specs=pl.BlockSpec((1, _CB, _NS), lambda b, j: (b, j, 0)),
        out_shape=jax.ShapeDtypeStruct((B, _NP, _NS), jnp.int32),
        scratch_shapes=[pltpu.VMEM((_CB, _N), f32)],
    )(dist)
    gidx_flat = gidx.reshape(-1)

    b1p = jnp.pad(b1, (0, 64))
    W2p = jnp.pad(W2, ((0, 64), (0, 0)))

    X1 = _sc_gather(G, gidx_flat)

    MB = 4096
    out = pl.pallas_call(
        _mlp_body,
        grid=(B * _NP * _NS // MB,),
        in_specs=[
            pl.BlockSpec((MB, 128), lambda i: (i, 0)),
            pl.BlockSpec((1, 128), lambda i: (0, 0)),
            pl.BlockSpec((128, 64), lambda i: (0, 0)),
            pl.BlockSpec((1, 64), lambda i: (0, 0)),
            pl.BlockSpec((64, 128), lambda i: (0, 0)),
            pl.BlockSpec((1, 128), lambda i: (0, 0)),
        ],
        out_specs=pl.BlockSpec((128, 128), lambda i: (i, 0)),
        out_shape=jax.ShapeDtypeStruct((B * _NP, 128), f32),
    )(X1, b1p.reshape(1, 128), W2p, b2.reshape(1, 64), W3, b3.reshape(1, 128))

    cent_xyz = jnp.stack([cx, cy, cz], axis=-1)
    return (cent_xyz, out.reshape(B, _NP, 128))


# speedup vs baseline: 21.4377x; 1.2810x over previous
import functools

import numpy as np
import jax
import jax.numpy as jnp
from jax import lax
from jax.experimental import pallas as pl
from jax.experimental.pallas import tpu as pltpu
from jax.experimental.pallas import tpu_sc as plsc

_B, _N, _DP = 8, 4096, 64
_NP = 512
_NS = 32
_CB = 128
_T04 = np.float32(0.2 ** 2)


def _fps_body(xyz_ref, init_ref, t_ref, w_ref, cx_ref, cy_ref, cz_ref,
              dist_ref, g_ref, dbuf0, dbuf1, dsem):
    g_ref[...] = lax.dot_general(
        t_ref[...], w_ref[...], (((1,), (0,)), ((), ())),
        preferred_element_type=jnp.float32)
    X = xyz_ref[:, 0, :]
    Y = xyz_ref[:, 1, :]
    Z = xyz_ref[:, 2, :]
    iota = lax.broadcasted_iota(jnp.int32, (_B, _N), 1)
    slot = lax.broadcasted_iota(jnp.int32, (_B, _NP), 1)
    zero = jnp.zeros((_B, _N), jnp.float32)
    zc = jnp.zeros((_B, _NP), jnp.float32)

    def pick(sel):
        px = jnp.sum(jnp.where(sel, X, zero), axis=1, keepdims=True)
        py = jnp.sum(jnp.where(sel, Y, zero), axis=1, keepdims=True)
        pz = jnp.sum(jnp.where(sel, Z, zero), axis=1, keepdims=True)
        return px, py, pz

    init_i = init_ref[...].astype(jnp.int32)
    px, py, pz = pick(iota == init_i)
    cxs = jnp.where(slot == 0, px, zc)
    cys = jnp.where(slot == 0, py, zc)
    czs = jnp.where(slot == 0, pz, zc)
    mask = jnp.ones((_B, _N), jnp.float32)

    def dist_row(px, py, pz):
        dx = X - px
        dy = Y - py
        dz = Z - pz
        return jnp.sqrt(dx * dx + dy * dy + dz * dz)

    bufs = (dbuf0, dbuf1)

    def drain(s, i):
        pltpu.make_async_copy(bufs[s], dist_ref.at[:, pl.ds(i, 1), :],
                              dsem.at[s]).wait()

    def emit(s, i, d):
        bufs[s][...] = jnp.minimum(d, _T04)[:, None, :]
        pltpu.make_async_copy(bufs[s], dist_ref.at[:, pl.ds(i, 1), :],
                              dsem.at[s]).start()

    def body(i, carry):
        px, py, pz, cxs, cys, czs, mask = carry
        d = dist_row(px, py, pz)
        par = lax.rem(i, 2)
        for s in (0, 1):
            @pl.when((par == s) & (i >= 2))
            def _(s=s):
                drain(s, i - 2)

            @pl.when(par == s)
            def _(s=s):
                emit(s, i, d)

        dm = d * mask
        mx = jnp.max(dm, axis=1, keepdims=True)
        idx = jnp.min(jnp.where(dm == mx, iota, jnp.int32(_N)), axis=1,
                      keepdims=True)
        npx, npy, npz = pick(iota == idx)
        nmask = jnp.minimum(dm * mask * jnp.float32(1e11), mask)
        w = slot == (i + 1)
        cxs = jnp.where(w, npx, cxs)
        cys = jnp.where(w, npy, cys)
        czs = jnp.where(w, npz, czs)
        return (npx, npy, npz, cxs, cys, czs, nmask)

    carry = (px, py, pz, cxs, cys, czs, mask)
    px, py, pz, cxs, cys, czs, _ = lax.fori_loop(0, _NP - 1, body, carry)
    drain(0, _NP - 2)
    drain(1, _NP - 3)
    emit(0, _NP - 1, dist_row(px, py, pz))
    drain(0, _NP - 1)
    cx_ref[...] = cxs
    cy_ref[...] = cys
    cz_ref[...] = czs


def _ballq_body(dist_ref, out_ref, key_scr):
    b = pl.program_id(0)
    j = pl.program_id(1)
    c0 = pl.multiple_of(j * _CB, 128)
    dc = dist_ref[0]
    iota = lax.broadcasted_iota(jnp.int32, (_CB, _N), 1)
    key = jnp.where(dc < _T04, dc, jnp.float32(1.0) + iota.astype(jnp.float32))
    key_scr[...] = key
    base = b * _N
    kcol = lax.broadcasted_iota(jnp.int32, (_CB, _NS), 1)
    one = jnp.float32(1.0)
    inf = jnp.float32(jnp.inf)

    def p1_cond(carry):
        _, _, mn = carry
        return jnp.min(mn) < one

    def p1_body(carry):
        acc, cnt, mn = carry
        key = key_scr[...]
        idx = jnp.min(jnp.where(key == mn[:, None], iota, jnp.int32(_N)),
                      axis=1)
        act = mn < one
        acc = jnp.where((kcol == cnt[:, None]) & act[:, None],
                        (idx + base)[:, None], acc)
        cnt = cnt + act.astype(jnp.int32)
        key = jnp.where((iota == idx[:, None]) & act[:, None], inf, key)
        key_scr[...] = key
        return (acc, cnt, jnp.min(key, axis=1))

    acc0 = jnp.zeros((_CB, _NS), jnp.int32)
    cnt0 = jnp.zeros((_CB,), jnp.int32)
    mn0 = jnp.min(key, axis=1)
    acc, cnt, _ = lax.while_loop(p1_cond, p1_body, (acc0, cnt0, mn0))

    iota64 = lax.broadcasted_iota(jnp.int32, (_CB, 64), 1)
    sub = key_scr[:, :64]
    av = (sub < jnp.float32(1e30)).astype(jnp.int32)
    pc = av
    for sh in (1, 2, 4, 8, 16, 32):
        pc = pc + jnp.concatenate(
            [jnp.zeros((_CB, sh), jnp.int32), pc[:, :64 - sh]], axis=1)
    slotv = cnt[:, None] + (pc - av)
    commit = (av > 0) & (slotv < _NS)
    val = iota64 + base
    big = jnp.int32(1 << 30)
    for s in range(_NS):
        v = jnp.min(jnp.where(commit & (slotv == s), val, big), axis=1)
        acc = jnp.where((kcol == s) & (v[:, None] < big), v[:, None], acc)
    out_ref[0, :, :] = acc


def _mm_body(t_ref, w_ref, out_ref):
    out_ref[...] = lax.dot_general(
        t_ref[...], w_ref[...], (((1,), (0,)), ((), ())),
        preferred_element_type=jnp.float32)


def _mlp_body(x_ref, b1_ref, w2_ref, b2_ref, w3_ref, b3_ref, out_ref):
    dn = (((1,), (0,)), ((), ()))
    h = jnp.maximum(x_ref[...] + b1_ref[...], jnp.float32(0.0))
    h = lax.dot_general(h, w2_ref[...], dn,
                        preferred_element_type=jnp.float32)
    h = jnp.maximum(h + b2_ref[...], jnp.float32(0.0))
    h = lax.dot_general(h, w3_ref[...], dn,
                        preferred_element_type=jnp.float32)
    h = jnp.maximum(h + b3_ref[...], jnp.float32(0.0))
    out_ref[...] = jnp.max(h.reshape(-1, _NS, 128), axis=1)


def _sc_gather(g, idx):
    info = plsc.get_sparse_core_info()
    ncores = info.num_cores
    nw = ncores * info.num_subcores
    rows = idx.shape[0]
    per_w = rows // nw
    chunk = 128
    nchunks = per_w // chunk
    mesh = plsc.VectorSubcoreMesh(core_axis_name="c", subcore_axis_name="s")

    @functools.partial(
        pl.kernel, mesh=mesh,
        out_type=jax.ShapeDtypeStruct((rows, g.shape[1]), jnp.float32),
        scratch_types=[
            pltpu.VMEM((chunk,), jnp.int32),
            pltpu.VMEM((chunk,), jnp.int32),
            pltpu.VMEM((chunk, g.shape[1]), jnp.float32),
            pltpu.VMEM((chunk, g.shape[1]), jnp.float32),
            pltpu.SemaphoreType.DMA,
            pltpu.SemaphoreType.DMA,
        ],
    )
    def k(g_hbm, idx_hbm, out_hbm, iv0, iv1, rv0, rv1, sg0, sg1):
        wid = lax.axis_index("s") * ncores + lax.axis_index("c")
        base = wid * per_w
        iv = (iv0, iv1)
        rv = (rv0, rv1)
        sg = (sg0, sg1)

        pltpu.sync_copy(idx_hbm.at[pl.ds(base, chunk)], iv0)
        copies = [pltpu.async_copy(g_hbm.at[iv0], rv0, sg0), None]
        pltpu.sync_copy(idx_hbm.at[pl.ds(base + chunk, chunk)], iv1)
        for c in range(nchunks):
            cur, nxt = c % 2, (c + 1) % 2
            copies[cur].wait()
            if c + 1 < nchunks:
                copies[nxt] = pltpu.async_copy(g_hbm.at[iv[nxt]], rv[nxt],
                                               sg[nxt])
            pltpu.sync_copy(rv[cur], out_hbm.at[pl.ds(base + c * chunk,
                                                      chunk)])
            if c + 2 < nchunks:
                pltpu.sync_copy(idx_hbm.at[pl.ds(base + (c + 2) * chunk,
                                                 chunk)], iv[cur])

    return k(g, idx)


def kernel(xyz, points, W1, b1, W2, b2, W3, b3):
    B, N, _ = xyz.shape
    f32 = jnp.float32
    init_id = jax.random.randint(jax.random.key(42), (B,), 0, N - 1)
    init_f = init_id.astype(f32).reshape(B, 1)
    xyz_t = jnp.transpose(xyz, (0, 2, 1))

    W1p = jnp.pad(W1, ((0, 0), (0, 64)))
    T = jnp.concatenate([xyz, points], axis=-1).reshape(B * N, 3 + _DP)
    cshape = jax.ShapeDtypeStruct((B, _NP), f32)
    cx, cy, cz, dist, G = pl.pallas_call(
        _fps_body,
        out_shape=[cshape, cshape, cshape,
                   jax.ShapeDtypeStruct((B, _NP, N), f32),
                   jax.ShapeDtypeStruct((B * N, 128), f32)],
        out_specs=[pl.BlockSpec(), pl.BlockSpec(), pl.BlockSpec(),
                   pl.BlockSpec(memory_space=pltpu.HBM), pl.BlockSpec()],
        scratch_shapes=[pltpu.VMEM((B, 1, N), f32),
                        pltpu.VMEM((B, 1, N), f32),
                        pltpu.SemaphoreType.DMA((2,))],
    )(xyz_t, init_f, T, W1p)

    gidx = pl.pallas_call(
        _ballq_body,
        grid=(B, _NP // _CB),
        in_specs=[pl.BlockSpec((1, _CB, N), lambda b, j: (b, j, 0))],
        out_specs=pl.BlockSpec((1, _CB, _NS), lambda b, j: (b, j, 0)),
        out_shape=jax.ShapeDtypeStruct((B, _NP, _NS), jnp.int32),
        scratch_shapes=[pltpu.VMEM((_CB, _N), f32)],
    )(dist)
    gidx_flat = gidx.reshape(-1)

    b1p = jnp.pad(b1, (0, 64))
    W2p = jnp.pad(W2, ((0, 64), (0, 0)))

    X1 = _sc_gather(G, gidx_flat)

    MB = 4096
    out = pl.pallas_call(
        _mlp_body,
        grid=(B * _NP * _NS // MB,),
        in_specs=[
            pl.BlockSpec((MB, 128), lambda i: (i, 0)),
            pl.BlockSpec((1, 128), lambda i: (0, 0)),
            pl.BlockSpec((128, 64), lambda i: (0, 0)),
            pl.BlockSpec((1, 64), lambda i: (0, 0)),
            pl.BlockSpec((64, 128), lambda i: (0, 0)),
            pl.BlockSpec((1, 128), lambda i: (0, 0)),
        ],
        out_specs=pl.BlockSpec((128, 128), lambda i: (i, 0)),
        out_shape=jax.ShapeDtypeStruct((B * _NP, 128), f32),
    )(X1, b1p.reshape(1, 128), W2p, b2.reshape(1, 64), W3, b3.reshape(1, 128))

    cent_xyz = jnp.stack([cx, cy, cz], axis=-1)
    return (cent_xyz, out.reshape(B, _NP, 128))


# speedup vs baseline: 22.6288x; 1.0556x over previous
import functools

import numpy as np
import jax
import jax.numpy as jnp
from jax import lax
from jax.experimental import pallas as pl
from jax.experimental.pallas import tpu as pltpu
from jax.experimental.pallas import tpu_sc as plsc

_B, _N, _DP = 8, 4096, 64
_NP = 512
_NS = 32
_CB = 128
_T04 = np.float32(0.2 ** 2)


def _fps_body(xyz_ref, init_ref, t_ref, w_ref, cx_ref, cy_ref, cz_ref,
              dist_ref, g_ref, dbuf0, dbuf1, dsem):
    g_ref[...] = lax.dot_general(
        t_ref[...], w_ref[...], (((1,), (0,)), ((), ())),
        preferred_element_type=jnp.float32)
    X = xyz_ref[:, 0, :]
    Y = xyz_ref[:, 1, :]
    Z = xyz_ref[:, 2, :]
    iota = lax.broadcasted_iota(jnp.int32, (_B, _N), 1)
    slot = lax.broadcasted_iota(jnp.int32, (_B, _NP), 1)
    zero = jnp.zeros((_B, _N), jnp.float32)
    zc = jnp.zeros((_B, _NP), jnp.float32)

    def pick(sel):
        px = jnp.sum(jnp.where(sel, X, zero), axis=1, keepdims=True)
        py = jnp.sum(jnp.where(sel, Y, zero), axis=1, keepdims=True)
        pz = jnp.sum(jnp.where(sel, Z, zero), axis=1, keepdims=True)
        return px, py, pz

    init_i = init_ref[...].astype(jnp.int32)
    px, py, pz = pick(iota == init_i)
    cxs = jnp.where(slot == 0, px, zc)
    cys = jnp.where(slot == 0, py, zc)
    czs = jnp.where(slot == 0, pz, zc)
    mask = jnp.ones((_B, _N), jnp.float32)

    def dist_row(px, py, pz):
        dx = X - px
        dy = Y - py
        dz = Z - pz
        return jnp.sqrt(dx * dx + dy * dy + dz * dz)

    bufs = (dbuf0, dbuf1)

    def drain(s, i):
        pltpu.make_async_copy(bufs[s], dist_ref.at[:, pl.ds(i, 1), :],
                              dsem.at[s]).wait()

    def emit(s, i, d):
        bufs[s][...] = jnp.minimum(d, _T04)[:, None, :]
        pltpu.make_async_copy(bufs[s], dist_ref.at[:, pl.ds(i, 1), :],
                              dsem.at[s]).start()

    def body(i, carry):
        px, py, pz, cxs, cys, czs, mask = carry
        d = dist_row(px, py, pz)
        par = lax.rem(i, 2)
        for s in (0, 1):
            @pl.when((par == s) & (i >= 2))
            def _(s=s):
                drain(s, i - 2)

            @pl.when(par == s)
            def _(s=s):
                emit(s, i, d)

        dm = d * mask
        mx = jnp.max(dm, axis=1, keepdims=True)
        idx = jnp.min(jnp.where(dm == mx, iota, jnp.int32(_N)), axis=1,
                      keepdims=True)
        npx, npy, npz = pick(iota == idx)
        nmask = jnp.minimum(dm * mask * jnp.float32(1e11), mask)
        w = slot == (i + 1)
        cxs = jnp.where(w, npx, cxs)
        cys = jnp.where(w, npy, cys)
        czs = jnp.where(w, npz, czs)
        return (npx, npy, npz, cxs, cys, czs, nmask)

    carry = (px, py, pz, cxs, cys, czs, mask)
    px, py, pz, cxs, cys, czs, _ = lax.fori_loop(0, _NP - 1, body, carry)
    drain(0, _NP - 2)
    drain(1, _NP - 3)
    emit(0, _NP - 1, dist_row(px, py, pz))
    drain(0, _NP - 1)
    cx_ref[...] = cxs
    cy_ref[...] = cys
    cz_ref[...] = czs


def _make_ballq(boff):
    def _ballq_body(dist_ref, out_ref, key_scr):
        b = pl.program_id(0) + boff
        j = pl.program_id(1)
        c0 = pl.multiple_of(j * _CB, 128)
        dc = dist_ref[0]
        iota = lax.broadcasted_iota(jnp.int32, (_CB, _N), 1)
        key = jnp.where(dc < _T04, dc, jnp.float32(1.0) + iota.astype(jnp.float32))
        key_scr[...] = key
        base = b * _N
        kcol = lax.broadcasted_iota(jnp.int32, (_CB, _NS), 1)
        one = jnp.float32(1.0)
        inf = jnp.float32(jnp.inf)

        def p1_cond(carry):
            _, _, mn = carry
            return jnp.min(mn) < one

        def p1_body(carry):
            acc, cnt, mn = carry
            key = key_scr[...]
            idx = jnp.min(jnp.where(key == mn[:, None], iota, jnp.int32(_N)),
                          axis=1)
            act = mn < one
            acc = jnp.where((kcol == cnt[:, None]) & act[:, None],
                            (idx + base)[:, None], acc)
            cnt = cnt + act.astype(jnp.int32)
            key = jnp.where((iota == idx[:, None]) & act[:, None], inf, key)
            key_scr[...] = key
            return (acc, cnt, jnp.min(key, axis=1))

        acc0 = jnp.zeros((_CB, _NS), jnp.int32)
        cnt0 = jnp.zeros((_CB,), jnp.int32)
        mn0 = jnp.min(key, axis=1)
        acc, cnt, _ = lax.while_loop(p1_cond, p1_body, (acc0, cnt0, mn0))

        iota64 = lax.broadcasted_iota(jnp.int32, (_CB, 64), 1)
        sub = key_scr[:, :64]
        av = (sub < jnp.float32(1e30)).astype(jnp.int32)
        pc = av
        for sh in (1, 2, 4, 8, 16, 32):
            pc = pc + jnp.concatenate(
                [jnp.zeros((_CB, sh), jnp.int32), pc[:, :64 - sh]], axis=1)
        slotv = cnt[:, None] + (pc - av)
        commit = (av > 0) & (slotv < _NS)
        val = iota64 + base
        big = jnp.int32(1 << 30)
        for s in range(_NS):
            v = jnp.min(jnp.where(commit & (slotv == s), val, big), axis=1)
            acc = jnp.where((kcol == s) & (v[:, None] < big), v[:, None], acc)
        out_ref[0, :, :] = acc
    return _ballq_body


def _mm_body(t_ref, w_ref, out_ref):
    out_ref[...] = lax.dot_general(
        t_ref[...], w_ref[...], (((1,), (0,)), ((), ())),
        preferred_element_type=jnp.float32)


def _mlp_body(x_ref, b1_ref, w2_ref, b2_ref, w3_ref, b3_ref, out_ref):
    dn = (((1,), (0,)), ((), ()))
    h = jnp.maximum(x_ref[...] + b1_ref[...], jnp.float32(0.0))
    h = lax.dot_general(h, w2_ref[...], dn,
                        preferred_element_type=jnp.float32)
    h = jnp.maximum(h + b2_ref[...], jnp.float32(0.0))
    h = lax.dot_general(h, w3_ref[...], dn,
                        preferred_element_type=jnp.float32)
    h = jnp.maximum(h + b3_ref[...], jnp.float32(0.0))
    out_ref[...] = jnp.max(h.reshape(-1, _NS, 128), axis=1)


def _sc_gather(g, idx):
    info = plsc.get_sparse_core_info()
    ncores = info.num_cores
    nw = ncores * info.num_subcores
    rows = idx.shape[0]
    per_w = rows // nw
    chunk = 128
    nchunks = per_w // chunk
    mesh = plsc.VectorSubcoreMesh(core_axis_name="c", subcore_axis_name="s")

    @functools.partial(
        pl.kernel, mesh=mesh,
        out_type=jax.ShapeDtypeStruct((rows, g.shape[1]), jnp.float32),
        scratch_types=[
            pltpu.VMEM((chunk,), jnp.int32),
            pltpu.VMEM((chunk,), jnp.int32),
            pltpu.VMEM((chunk, g.shape[1]), jnp.float32),
            pltpu.VMEM((chunk, g.shape[1]), jnp.float32),
            pltpu.SemaphoreType.DMA,
            pltpu.SemaphoreType.DMA,
        ],
    )
    def k(g_hbm, idx_hbm, out_hbm, iv0, iv1, rv0, rv1, sg0, sg1):
        wid = lax.axis_index("s") * ncores + lax.axis_index("c")
        base = wid * per_w
        iv = (iv0, iv1)
        rv = (rv0, rv1)
        sg = (sg0, sg1)

        pltpu.sync_copy(idx_hbm.at[pl.ds(base, chunk)], iv0)
        copies = [pltpu.async_copy(g_hbm.at[iv0], rv0, sg0), None]
        pltpu.sync_copy(idx_hbm.at[pl.ds(base + chunk, chunk)], iv1)
        for c in range(nchunks):
            cur, nxt = c % 2, (c + 1) % 2
            copies[cur].wait()
            if c + 1 < nchunks:
                copies[nxt] = pltpu.async_copy(g_hbm.at[iv[nxt]], rv[nxt],
                                               sg[nxt])
            pltpu.sync_copy(rv[cur], out_hbm.at[pl.ds(base + c * chunk,
                                                      chunk)])
            if c + 2 < nchunks:
                pltpu.sync_copy(idx_hbm.at[pl.ds(base + (c + 2) * chunk,
                                                 chunk)], iv[cur])

    return k(g, idx)


def kernel(xyz, points, W1, b1, W2, b2, W3, b3):
    B, N, _ = xyz.shape
    f32 = jnp.float32
    init_id = jax.random.randint(jax.random.key(42), (B,), 0, N - 1)
    init_f = init_id.astype(f32).reshape(B, 1)
    xyz_t = jnp.transpose(xyz, (0, 2, 1))

    W1p = jnp.pad(W1, ((0, 0), (0, 64)))
    T = jnp.concatenate([xyz, points], axis=-1).reshape(B * N, 3 + _DP)
    cshape = jax.ShapeDtypeStruct((B, _NP), f32)
    cx, cy, cz, dist, G = pl.pallas_call(
        _fps_body,
        out_shape=[cshape, cshape, cshape,
                   jax.ShapeDtypeStruct((B, _NP, N), f32),
                   jax.ShapeDtypeStruct((B * N, 128), f32)],
        out_specs=[pl.BlockSpec(), pl.BlockSpec(), pl.BlockSpec(),
                   pl.BlockSpec(memory_space=pltpu.HBM), pl.BlockSpec()],
        scratch_shapes=[pltpu.VMEM((B, 1, N), f32),
                        pltpu.VMEM((B, 1, N), f32),
                        pltpu.SemaphoreType.DMA((2,))],
    )(xyz_t, init_f, T, W1p)

    b1p = jnp.pad(b1, (0, 64))
    W2p = jnp.pad(W2, ((0, 64), (0, 0)))

    HB = B // 2
    MB = 4096
    outs = []
    for h in (0, 1):
        gidx = pl.pallas_call(
            _make_ballq(h * HB),
            grid=(HB, _NP // _CB),
            in_specs=[pl.BlockSpec((1, _CB, N),
                                   lambda b, j, h=h: (b + h * HB, j, 0))],
            out_specs=pl.BlockSpec((1, _CB, _NS), lambda b, j: (b, j, 0)),
            out_shape=jax.ShapeDtypeStruct((HB, _NP, _NS), jnp.int32),
            scratch_shapes=[pltpu.VMEM((_CB, _N), f32)],
        )(dist)
        X1 = _sc_gather(G, gidx.reshape(-1))
        out_h = pl.pallas_call(
            _mlp_body,
            grid=(HB * _NP * _NS // MB,),
            in_specs=[
                pl.BlockSpec((MB, 128), lambda i: (i, 0)),
                pl.BlockSpec((1, 128), lambda i: (0, 0)),
                pl.BlockSpec((128, 64), lambda i: (0, 0)),
                pl.BlockSpec((1, 64), lambda i: (0, 0)),
                pl.BlockSpec((64, 128), lambda i: (0, 0)),
                pl.BlockSpec((1, 128), lambda i: (0, 0)),
            ],
            out_specs=pl.BlockSpec((128, 128), lambda i: (i, 0)),
            out_shape=jax.ShapeDtypeStruct((HB * _NP, 128), f32),
        )(X1, b1p.reshape(1, 128), W2p, b2.reshape(1, 64), W3,
          b3.reshape(1, 128))
        outs.append(out_h)

    out = jnp.concatenate(outs, axis=0)
    cent_xyz = jnp.stack([cx, cy, cz], axis=-1)
    return (cent_xyz, out.reshape(B, _NP, 128))
